# Initial kernel scaffold; baseline (speedup 1.0000x reference)
import jax
import jax.numpy as jnp
from jax.experimental import pallas as pl


def kernel(x, edge_index, gcn_W, gcn_b, pre_W, pre_b, ln_g, ln_b, rt_W, rt_b, cls_W, cls_b):
    raise NotImplementedError("write your pallas kernel here")


# speedup vs baseline: 1.0959x; 1.0959x over previous
import functools
import jax
import jax.numpy as jnp
from jax.experimental import pallas as pl
from jax.experimental.pallas import tpu as pltpu

N = 10000
E = 320000
F_IN = 128
H = 64
CAP = 18
C = 7
ITERS = 3


def _matmul_kernel(x_ref, w_ref, o_ref):
    o_ref[...] = jnp.dot(x_ref[...], w_ref[...],
                         preferred_element_type=jnp.float32)


def _tc_matmul(x, w):
    m, k = x.shape
    _, n = w.shape
    bm = 512
    grid = (pl.cdiv(m, bm),)
    return pl.pallas_call(
        _matmul_kernel,
        grid=grid,
        in_specs=[
            pl.BlockSpec((bm, k), lambda i: (i, 0)),
            pl.BlockSpec((k, n), lambda i: (0, 0)),
        ],
        out_specs=pl.BlockSpec((bm, n), lambda i: (i, 0)),
        out_shape=jax.ShapeDtypeStruct((m, n), jnp.float32),
    )(x, w)


def _segment_softmax(logits, seg, num_segments):
    m = jax.ops.segment_max(logits, seg, num_segments=num_segments)
    m = jnp.where(jnp.isfinite(m), m, 0.0)
    ex = jnp.exp(logits - m[seg])
    s = jax.ops.segment_sum(ex, seg, num_segments=num_segments)
    return ex / (s[seg] + 1e-16)


def _squash(s):
    norm_sq = jnp.sum(s ** 2, axis=-1, keepdims=True)
    scale = norm_sq / (1.0 + norm_sq)
    norm = jnp.sqrt(norm_sq + 1e-09)
    return scale * (s / norm)


def kernel(x, edge_index, gcn_W, gcn_b, pre_W, pre_b, ln_g, ln_b, rt_W, rt_b, cls_W, cls_b):
    src = edge_index[0]
    dst = edge_index[1]
    loops = jnp.arange(N, dtype=edge_index.dtype)
    src2 = jnp.concatenate([src, loops])
    dst2 = jnp.concatenate([dst, loops])
    deg = jax.ops.segment_sum(jnp.ones_like(src2, dtype=jnp.float32), dst2, num_segments=N)
    dinv = jnp.where(deg > 0, 1.0 / jnp.sqrt(deg), 0.0)
    enorm = dinv[src2] * dinv[dst2]
    h = _tc_matmul(x, gcn_W)
    h = jax.ops.segment_sum(enorm[:, None] * h[src2], dst2, num_segments=N) + gcn_b
    h = jax.nn.relu(h)
    h = h @ pre_W + pre_b
    mu = jnp.mean(h, axis=-1, keepdims=True)
    var = jnp.var(h, axis=-1, keepdims=True)
    h = (h - mu) / jnp.sqrt(var + 1e-5) * ln_g + ln_b
    u_hat = h @ rt_W + rt_b
    u_e = u_hat[src]
    b = jnp.zeros((E,), dtype=jnp.float32)
    v = None
    for r in range(ITERS):
        c = _segment_softmax(b, dst, N)
        v = jax.ops.segment_sum(c[:, None] * u_e, dst, num_segments=N)
        v = _squash(v)
        if r < ITERS - 1:
            b = b + jnp.sum(v[dst] * u_e, axis=-1)
    return v @ cls_W + cls_b


# speedup vs baseline: 10.2661x; 9.3675x over previous
import functools
import jax
import jax.numpy as jnp
import numpy as np
from jax import lax
from jax.experimental import pallas as pl
from jax.experimental.pallas import tpu as pltpu
from jax.experimental.pallas import tpu_sc as plsc

N = 10000
E = 320000
F_IN = 128
H = 64
CAP = 18
C = 7
ITERS = 3

NC, NS, L = 2, 16, 16
NW = NC * NS
N_PAD = 10240
E_PAD = 327680
G = 128
NG = E_PAD // (NW * G)
NROW = N_PAD // 16
CAP_PAD = 32

_SC_PARAMS = pltpu.CompilerParams(
    use_tc_tiling_on_sc=False, needs_layout_passes=False)
_MESH = plsc.VectorSubcoreMesh(core_axis_name="c", subcore_axis_name="s")

IOTA16 = lambda: lax.iota(jnp.int32, 16)


def _rsqrt_nt(x):
    bits = lax.bitcast_convert_type(x, jnp.int32)
    magic = jnp.full(x.shape, 0x5F3759DF, jnp.int32)
    y = lax.bitcast_convert_type(magic - lax.shift_right_arithmetic(bits, 1),
                                 jnp.float32)
    for _ in range(4):
        y = y * (1.5 - 0.5 * x * y * y)
    return y


def _wid():
    return lax.axis_index("s") * NC + lax.axis_index("c")


def _mm_kernel(x_ref, w_ref, o_ref):
    o_ref[...] = jnp.dot(x_ref[...], w_ref[...],
                         preferred_element_type=jnp.float32)


def _tc_matmul(x, w, bm=1024):
    m, k = x.shape
    _, n = w.shape
    return pl.pallas_call(
        _mm_kernel,
        grid=(m // bm,),
        in_specs=[pl.BlockSpec((bm, k), lambda i: (i, 0)),
                  pl.BlockSpec((k, n), lambda i: (0, 0))],
        out_specs=pl.BlockSpec((bm, n), lambda i: (i, 0)),
        out_shape=jax.ShapeDtypeStruct((m, n), jnp.float32),
    )(x, w)


def _mid_kernel(h0_ref, h1_ref, dinvb_ref, g2_ref, gb_ref, preW_ref, preb_ref,
                lng_ref, lnb_ref, rtW_ref, rtb_ref, o_ref):
    hs = h0_ref[...] + h1_ref[...]
    h = jax.nn.relu(dinvb_ref[...] * hs + g2_ref[...] + gb_ref[...])
    hp = jnp.dot(h, preW_ref[...], preferred_element_type=jnp.float32)
    hp = hp + preb_ref[...]
    mu = jnp.sum(hp, axis=-1, keepdims=True) * (1.0 / CAP)
    m2 = jnp.sum(hp * hp, axis=-1, keepdims=True) * (1.0 / CAP)
    var = m2 - mu * mu
    xln = (hp - mu) * lax.rsqrt(var + 1e-5) * lng_ref[...] + lnb_ref[...]
    u = jnp.dot(xln, rtW_ref[...], preferred_element_type=jnp.float32)
    o_ref[...] = u + rtb_ref[...]


def _tc_middle(h0, h1, dinvb, g2, gb, preW, preb, lng, lnb, rtW, rtb, bm=1024):
    m = h0.shape[0]
    row64 = lambda i: (i, 0)
    fixed = lambda i: (0, 0)
    return pl.pallas_call(
        _mid_kernel,
        grid=(m // bm,),
        in_specs=[
            pl.BlockSpec((bm, H), row64), pl.BlockSpec((bm, H), row64),
            pl.BlockSpec((bm, H), row64), pl.BlockSpec((bm, H), row64),
            pl.BlockSpec((1, H), fixed),
            pl.BlockSpec((H, 128), fixed), pl.BlockSpec((1, 128), fixed),
            pl.BlockSpec((1, 128), fixed), pl.BlockSpec((1, 128), fixed),
            pl.BlockSpec((128, 128), fixed), pl.BlockSpec((1, 128), fixed),
        ],
        out_specs=pl.BlockSpec((bm, 128), row64),
        out_shape=jax.ShapeDtypeStruct((m, 128), jnp.float32),
    )(h0, h1, dinvb, g2, gb, preW, preb, lng, lnb, rtW, rtb)


@functools.partial(
    pl.kernel,
    out_type=jax.ShapeDtypeStruct((NW, NROW, 16), jnp.float32),
    mesh=_MESH, compiler_params=_SC_PARAMS,
    scratch_types=[
        pltpu.VMEM((NG, G), jnp.int32),
        pltpu.VMEM((NROW, 16), jnp.float32),
    ],
)
def _sc_deg(dst_hbm, deg_part, dstb, degl):
    w = _wid()
    pltpu.sync_copy(dst_hbm.at[w], dstb)

    @pl.loop(0, NROW)
    def _(j):
        degl[j, :] = jnp.zeros((16,), jnp.float32)

    onev = jnp.ones((16,), jnp.float32)

    @pl.loop(0, NG)
    def _(g):
        for i in range(G // 16):
            idx = dstb[g, pl.ds(i * 16, 16)]
            plsc.addupdate_scatter(degl, [idx >> 4, idx & 15], onev)

    pltpu.sync_copy(degl, deg_part.at[w])


NPW = N_PAD // NW
RPW = NROW // NW

@functools.partial(
    pl.kernel,
    out_type=[
        jax.ShapeDtypeStruct((NROW, 16), jnp.float32),
        jax.ShapeDtypeStruct((N_PAD, H), jnp.float32),
        jax.ShapeDtypeStruct((N_PAD, H), jnp.float32),
        jax.ShapeDtypeStruct((N_PAD, H), jnp.float32),
    ],
    mesh=_MESH, compiler_params=_SC_PARAMS,
    scratch_types=[
        pltpu.VMEM((RPW, 16), jnp.float32),
        pltpu.VMEM((RPW, 16), jnp.float32),
        pltpu.VMEM((RPW, 16), jnp.float32),
        pltpu.VMEM((NPW, H), jnp.float32),
        pltpu.VMEM((NPW, H), jnp.float32),
        pltpu.VMEM((NPW, H), jnp.float32),
    ],
)
def _sc_nodeprep(deg_part, hw_hbm, s0_out, g_out, g2_out, dinvb_out,
                 acc, tmp, dnv, hbuf, h2buf, h3buf):
    w = _wid()
    for j in range(RPW):
        acc[j, :] = jnp.zeros((16,), jnp.float32)

    @pl.loop(0, NW)
    def _(w2):
        pltpu.sync_copy(deg_part.at[w2, pl.ds(w * RPW, RPW)], tmp)
        for j in range(RPW):
            acc[j, :] = acc[j, :] + tmp[j, :]

    for j in range(RPW):
        dnv[j, :] = _rsqrt_nt(acc[j, :] + 1.0)
    pltpu.sync_copy(acc, s0_out.at[pl.ds(w * RPW, RPW)])

    pltpu.sync_copy(hw_hbm.at[pl.ds(w * NPW, NPW)], hbuf)

    @pl.loop(0, NPW // 16)
    def _(t):
        n16 = IOTA16() + t * 16
        dvv = dnv[t, :]
        for k in range(H):
            kv = jnp.full((16,), k, jnp.int32)
            row = plsc.load_gather(hbuf, [n16, kv])
            gg = row * dvv
            plsc.store_scatter(hbuf, [n16, kv], gg)
            plsc.store_scatter(h2buf, [n16, kv], gg * dvv)
            plsc.store_scatter(h3buf, [n16, kv], dvv)

    pltpu.sync_copy(hbuf, g_out.at[pl.ds(w * NPW, NPW)])
    pltpu.sync_copy(h2buf, g2_out.at[pl.ds(w * NPW, NPW)])
    pltpu.sync_copy(h3buf, dinvb_out.at[pl.ds(w * NPW, NPW)])


NPS = N_PAD // NS
ZCH = 160

@functools.partial(
    pl.kernel,
    out_type=jax.ShapeDtypeStruct((NC, N_PAD, H), jnp.float32),
    mesh=_MESH, compiler_params=_SC_PARAMS,
    scratch_types=[
        pltpu.VMEM((NG, G), jnp.int32),
        pltpu.VMEM((NG, G), jnp.int32),
        pltpu.VMEM((G, H), jnp.float32),
        pltpu.VMEM((G, H), jnp.float32),
        pltpu.VMEM((ZCH, H), jnp.float32),
        pltpu.VMEM_SHARED((N_PAD, H), jnp.float32),
        pltpu.SemaphoreType.DMA,
        pltpu.SemaphoreType.DMA,
    ],
)
def _sc_gcn(src_hbm, dst_hbm, g_hbm, hsum_part,
            sb, db, rows0, rows1, zb, hsum_sh, sem0, sem1):
    w = _wid()
    cid = lax.axis_index("c")
    sid = lax.axis_index("s")
    pltpu.sync_copy(src_hbm.at[w], sb)
    pltpu.sync_copy(dst_hbm.at[w], db)

    @pl.loop(0, ZCH)
    def _(j):
        for k in range(H // 16):
            zb[j, pl.ds(k * 16, 16)] = jnp.zeros((16,), jnp.float32)

    for z in range(NPS // ZCH):
        pltpu.sync_copy(zb, hsum_sh.at[pl.ds(sid * NPS + z * ZCH, ZCH)])
    plsc.subcore_barrier()

    @pl.loop(0, NG // 2)
    def _(t):
        g0 = t * 2
        g1 = g0 + 1
        cp0 = pltpu.async_copy(g_hbm.at[sb.at[g0]], rows0, sem0)
        cp1 = pltpu.async_copy(g_hbm.at[sb.at[g1]], rows1, sem1)
        cp0.wait()
        pltpu.sync_copy(rows0, hsum_sh.at[db.at[g0]], add=True)
        cp1.wait()
        pltpu.sync_copy(rows1, hsum_sh.at[db.at[g1]], add=True)

    plsc.subcore_barrier()
    for z in range(NPS // ZCH):
        sl = pl.ds(sid * NPS + z * ZCH, ZCH)
        pltpu.sync_copy(hsum_sh.at[sl], hsum_part.at[cid, sl])


def _make_sc_vpass(first):
    scratch = [
        pltpu.VMEM((NG, G), jnp.int32),
        pltpu.VMEM((NG, G), jnp.int32),
        pltpu.VMEM((NG, G), jnp.float32),
        pltpu.VMEM((NROW, 16), jnp.float32),
        pltpu.VMEM((NROW, 16), jnp.float32),
        pltpu.VMEM((G, CAP_PAD), jnp.float32),
        pltpu.VMEM((G, CAP_PAD), jnp.float32),
        pltpu.VMEM((G, CAP_PAD), jnp.float32),
        pltpu.VMEM((G, CAP_PAD), jnp.float32),
        pltpu.VMEM((ZCH, CAP_PAD), jnp.float32),
        pltpu.SemaphoreType.DMA,
        pltpu.SemaphoreType.DMA,
    ]

    @functools.partial(
        pl.kernel,
        out_type=jax.ShapeDtypeStruct((NC, N_PAD, CAP_PAD), jnp.float32),
        mesh=_MESH, compiler_params=_SC_PARAMS,
        scratch_types=scratch + [
            pltpu.VMEM_SHARED((N_PAD, CAP_PAD), jnp.float32)],
    )
    def vpass(src_hbm, dst_hbm, u_hbm, s_a, s_b, b_hbm, v_part,
              sb, db, bb, sfull, stmp, ub0, ub1, cu0, cu1, zb,
              sem0, sem1, v_sh):
        w = _wid()
        cid = lax.axis_index("c")
        sid = lax.axis_index("s")
        pltpu.sync_copy(src_hbm.at[w], sb)
        pltpu.sync_copy(dst_hbm.at[w], db)

        if first:
            pltpu.sync_copy(s_a, sfull)
        else:
            pltpu.sync_copy(s_a, sfull)
            pltpu.sync_copy(s_b, stmp)

            @pl.loop(0, NROW)
            def _(j):
                sfull[j, :] = sfull[j, :] + stmp[j, :]
            pltpu.sync_copy(b_hbm.at[w], bb)

        @pl.loop(0, ZCH)
        def _(j):
            for k in range(CAP_PAD // 16):
                zb[j, pl.ds(k * 16, 16)] = jnp.zeros((16,), jnp.float32)

        for z in range(NPS // ZCH):
            pltpu.sync_copy(zb, v_sh.at[pl.ds(sid * NPS + z * ZCH, ZCH)])

        @pl.loop(0, G)
        def _(j):
            for k in range(CAP_PAD // 16):
                cu0[j, pl.ds(k * 16, 16)] = jnp.zeros((16,), jnp.float32)
                cu1[j, pl.ds(k * 16, 16)] = jnp.zeros((16,), jnp.float32)
        plsc.subcore_barrier()

        def compute(g, ub, cu):
            for i in range(G // 16):
                dv = db[g, pl.ds(i * 16, 16)]
                sg = plsc.load_gather(sfull, [dv >> 4, dv & 15])
                if first:
                    c = 1.0 / (sg + 1e-16)
                else:
                    bv = bb[g, pl.ds(i * 16, 16)]
                    c = jnp.exp(bv) / (sg + 1e-16)
                ev = IOTA16() + (i * 16)
                for k in range(CAP):
                    kv = jnp.full((16,), k, jnp.int32)
                    uk = plsc.load_gather(ub, [ev, kv])
                    plsc.store_scatter(cu, [ev, kv], uk * c)

        @pl.loop(0, NG // 2)
        def _(t):
            g0 = t * 2
            g1 = g0 + 1
            cp0 = pltpu.async_copy(u_hbm.at[sb.at[g0]], ub0, sem0)
            cp1 = pltpu.async_copy(u_hbm.at[sb.at[g1]], ub1, sem1)
            cp0.wait()
            compute(g0, ub0, cu0)
            pltpu.sync_copy(cu0, v_sh.at[db.at[g0]], add=True)
            cp1.wait()
            compute(g1, ub1, cu1)
            pltpu.sync_copy(cu1, v_sh.at[db.at[g1]], add=True)

        plsc.subcore_barrier()
        for z in range(NPS // ZCH):
            sl = pl.ds(sid * NPS + z * ZCH, ZCH)
            pltpu.sync_copy(v_sh.at[sl], v_part.at[cid, sl])

    return vpass


_sc_vpass_first = _make_sc_vpass(True)
_sc_vpass_later = _make_sc_vpass(False)


@functools.partial(
    pl.kernel,
    out_type=jax.ShapeDtypeStruct((N_PAD, CAP_PAD), jnp.float32),
    mesh=_MESH, compiler_params=_SC_PARAMS,
    scratch_types=[
        pltpu.VMEM((NPW, CAP_PAD), jnp.float32),
        pltpu.VMEM((NPW, CAP_PAD), jnp.float32),
    ],
)
def _sc_squash(v_part, v_out, va, vb):
    w = _wid()
    sl = pl.ds(w * NPW, NPW)
    pltpu.sync_copy(v_part.at[0, sl], va)
    pltpu.sync_copy(v_part.at[1, sl], vb)

    @pl.loop(0, NPW // 16)
    def _(t):
        n16 = IOTA16() + t * 16
        nsq = jnp.zeros((16,), jnp.float32)
        for k in range(CAP):
            kv = jnp.full((16,), k, jnp.int32)
            s = plsc.load_gather(va, [n16, kv]) + plsc.load_gather(vb, [n16, kv])
            plsc.store_scatter(va, [n16, kv], s)
            nsq = nsq + s * s
        f = nsq / (1.0 + nsq) * _rsqrt_nt(nsq + 1e-9)
        for k in range(CAP):
            kv = jnp.full((16,), k, jnp.int32)
            s = plsc.load_gather(va, [n16, kv])
            plsc.store_scatter(va, [n16, kv], s * f)

    pltpu.sync_copy(va, v_out.at[sl])


def _make_sc_bpass(first):
    @functools.partial(
        pl.kernel,
        out_type=[
            jax.ShapeDtypeStruct((NW, NG, G), jnp.float32),
            jax.ShapeDtypeStruct((NC, NROW, 16), jnp.float32),
        ],
        mesh=_MESH, compiler_params=_SC_PARAMS,
        scratch_types=[
            pltpu.VMEM((NG, G), jnp.int32),
            pltpu.VMEM((NG, G), jnp.int32),
            pltpu.VMEM((NG, G), jnp.float32),
            pltpu.VMEM((NROW, 16), jnp.float32),
            pltpu.VMEM((G, CAP_PAD), jnp.float32),
            pltpu.VMEM((G, CAP_PAD), jnp.float32),
            pltpu.VMEM((G, CAP_PAD), jnp.float32),
            pltpu.VMEM((G, CAP_PAD), jnp.float32),
            pltpu.VMEM((NROW // NS, 16), jnp.float32),
            pltpu.VMEM((NROW // NS, 16), jnp.float32),
            pltpu.VMEM_SHARED((NS, NROW, 16), jnp.float32),
            pltpu.SemaphoreType.DMA,
            pltpu.SemaphoreType.DMA,
            pltpu.SemaphoreType.DMA,
            pltpu.SemaphoreType.DMA,
        ],
    )
    def bpass(src_hbm, dst_hbm, u_hbm, v_hbm, b_in, b_out, s_part,
              sb, db, bb, sloc, ub0, vb0, ub1, vb1, acc, tmp, s_sh,
              sem0, sem1, sem2, sem3):
        w = _wid()
        cid = lax.axis_index("c")
        sid = lax.axis_index("s")
        pltpu.sync_copy(src_hbm.at[w], sb)
        pltpu.sync_copy(dst_hbm.at[w], db)
        if not first:
            pltpu.sync_copy(b_in.at[w], bb)

        @pl.loop(0, NROW)
        def _(j):
            sloc[j, :] = jnp.zeros((16,), jnp.float32)

        def compute(g, ub, vb):
            for i in range(G // 16):
                ev = IOTA16() + (i * 16)
                accv = jnp.zeros((16,), jnp.float32)
                for k in range(CAP):
                    kv = jnp.full((16,), k, jnp.int32)
                    uk = plsc.load_gather(ub, [ev, kv])
                    vk = plsc.load_gather(vb, [ev, kv])
                    accv = accv + uk * vk
                if first:
                    bnew = accv
                else:
                    bnew = bb[g, pl.ds(i * 16, 16)] + accv
                bb[g, pl.ds(i * 16, 16)] = bnew
                ex = jnp.exp(bnew)
                dv = db[g, pl.ds(i * 16, 16)]
                plsc.addupdate_scatter(sloc, [dv >> 4, dv & 15], ex)

        @pl.loop(0, NG // 2)
        def _(t):
            g0 = t * 2
            g1 = g0 + 1
            cu0 = pltpu.async_copy(u_hbm.at[sb.at[g0]], ub0, sem0)
            cv0 = pltpu.async_copy(v_hbm.at[db.at[g0]], vb0, sem1)
            cu1 = pltpu.async_copy(u_hbm.at[sb.at[g1]], ub1, sem2)
            cv1 = pltpu.async_copy(v_hbm.at[db.at[g1]], vb1, sem3)
            cu0.wait()
            cv0.wait()
            compute(g0, ub0, vb0)
            cu1.wait()
            cv1.wait()
            compute(g1, ub1, vb1)

        pltpu.sync_copy(bb, b_out.at[w])
        pltpu.sync_copy(sloc, s_sh.at[sid])
        plsc.subcore_barrier()

        rps = NROW // NS
        for j in range(rps):
            acc[j, :] = jnp.zeros((16,), jnp.float32)

        @pl.loop(0, NS)
        def _(j):
            pltpu.sync_copy(s_sh.at[j, pl.ds(sid * rps, rps)], tmp)
            for r in range(rps):
                acc[r, :] = acc[r, :] + tmp[r, :]

        pltpu.sync_copy(acc, s_part.at[cid, pl.ds(sid * rps, rps)])

    return bpass


_sc_bpass_first = _make_sc_bpass(True)
_sc_bpass_later = _make_sc_bpass(False)


def kernel(x, edge_index, gcn_W, gcn_b, pre_W, pre_b, ln_g, ln_b, rt_W, rt_b,
           cls_W, cls_b):
    src = edge_index[0]
    dst = edge_index[1]
    padi = jnp.full((E_PAD - E,), N_PAD - 1, dtype=src.dtype)
    srcp = jnp.concatenate([src, padi]).reshape(NW, NG, G).astype(jnp.int32)
    dstp = jnp.concatenate([dst, padi]).reshape(NW, NG, G).astype(jnp.int32)

    xp = jnp.pad(x, ((0, N_PAD - N), (0, 0)))
    hW = _tc_matmul(xp, gcn_W)

    deg_part = _sc_deg(dstp)
    s0, g_tab, g2_tab, dinvb = _sc_nodeprep(deg_part, hW)

    hsum_part = _sc_gcn(srcp, dstp, g_tab)

    preWp = jnp.pad(pre_W, ((0, 0), (0, 128 - CAP)))
    prebp = jnp.pad(pre_b, (0, 128 - CAP)).reshape(1, 128)
    lngp = jnp.pad(ln_g, (0, 128 - CAP)).reshape(1, 128)
    lnbp = jnp.pad(ln_b, (0, 128 - CAP)).reshape(1, 128)
    rtWp = jnp.pad(rt_W, ((0, 128 - CAP), (0, 128 - CAP)))
    rtbp = jnp.pad(rt_b, (0, 128 - CAP)).reshape(1, 128)
    gbp = jnp.broadcast_to(gcn_b.reshape(1, H), (1, H))

    u_full = _tc_middle(hsum_part[0], hsum_part[1], dinvb, g2_tab, gbp,
                        preWp, prebp, lngp, lnbp, rtWp, rtbp)
    u32 = u_full[:, :CAP_PAD]

    dummy_b = jnp.zeros((NW, NG, G), jnp.float32)
    dummy_s = jnp.zeros((NROW, 16), jnp.float32)

    v_part = _sc_vpass_first(srcp, dstp, u32, s0, dummy_s, dummy_b)
    v = _sc_squash(v_part)
    b, s_part = _sc_bpass_first(srcp, dstp, u32, v, dummy_b)

    v_part = _sc_vpass_later(srcp, dstp, u32, s_part[0], s_part[1], b)
    v = _sc_squash(v_part)
    b, s_part = _sc_bpass_later(srcp, dstp, u32, v, b)

    v_part = _sc_vpass_later(srcp, dstp, u32, s_part[0], s_part[1], b)
    v = _sc_squash(v_part)

    clsWp = jnp.pad(cls_W, ((0, CAP_PAD - CAP), (0, 128 - C)))
    clsbp = jnp.pad(cls_b, (0, 128 - C)).reshape(1, 128)
    out = pl.pallas_call(
        _mm_kernel,
        grid=(N_PAD // 1024,),
        in_specs=[pl.BlockSpec((1024, CAP_PAD), lambda i: (i, 0)),
                  pl.BlockSpec((CAP_PAD, 128), lambda i: (0, 0))],
        out_specs=pl.BlockSpec((1024, 128), lambda i: (i, 0)),
        out_shape=jax.ShapeDtypeStruct((N_PAD, 128), jnp.float32),
    )(v, clsWp)
    out = out + clsbp
    return out[:N, :C]


# speedup vs baseline: 13.4966x; 1.3147x over previous
import functools
import jax
import jax.numpy as jnp
import numpy as np
from jax import lax
from jax.experimental import pallas as pl
from jax.experimental.pallas import tpu as pltpu
from jax.experimental.pallas import tpu_sc as plsc

N = 10000
E = 320000
F_IN = 128
H = 64
CAP = 18
C = 7
ITERS = 3

NC, NS, L = 2, 16, 16
NW = NC * NS
N_PAD = 10240
E_PAD = 327680
G = 128
NG = E_PAD // (NW * G)
NROW = N_PAD // 16
CAP_PAD = 32

_SC_PARAMS = pltpu.CompilerParams(
    use_tc_tiling_on_sc=False, needs_layout_passes=False)
_MESH = plsc.VectorSubcoreMesh(core_axis_name="c", subcore_axis_name="s")

IOTA16 = lambda: lax.iota(jnp.int32, 16)


def _rsqrt_nt(x):
    bits = lax.bitcast_convert_type(x, jnp.int32)
    magic = jnp.full(x.shape, 0x5F3759DF, jnp.int32)
    y = lax.bitcast_convert_type(magic - lax.shift_right_arithmetic(bits, 1),
                                 jnp.float32)
    for _ in range(4):
        y = y * (1.5 - 0.5 * x * y * y)
    return y


def _wid():
    return lax.axis_index("s") * NC + lax.axis_index("c")


def _mm_kernel(x_ref, w_ref, o_ref):
    o_ref[...] = jnp.dot(x_ref[...], w_ref[...],
                         preferred_element_type=jnp.float32)


def _tc_matmul(x, w, bm=1024):
    m, k = x.shape
    _, n = w.shape
    return pl.pallas_call(
        _mm_kernel,
        grid=(m // bm,),
        in_specs=[pl.BlockSpec((bm, k), lambda i: (i, 0)),
                  pl.BlockSpec((k, n), lambda i: (0, 0))],
        out_specs=pl.BlockSpec((bm, n), lambda i: (i, 0)),
        out_shape=jax.ShapeDtypeStruct((m, n), jnp.float32),
    )(x, w)


def _mid_kernel(h0_ref, h1_ref, dinvb_ref, g2_ref, gb_ref, preW_ref, preb_ref,
                lng_ref, lnb_ref, rtW_ref, rtb_ref, o_ref):
    hs = h0_ref[...] + h1_ref[...]
    h = jax.nn.relu(dinvb_ref[...] * hs + g2_ref[...] + gb_ref[...])
    hp = jnp.dot(h, preW_ref[...], preferred_element_type=jnp.float32)
    hp = hp + preb_ref[...]
    mu = jnp.sum(hp, axis=-1, keepdims=True) * (1.0 / CAP)
    m2 = jnp.sum(hp * hp, axis=-1, keepdims=True) * (1.0 / CAP)
    var = m2 - mu * mu
    xln = (hp - mu) * lax.rsqrt(var + 1e-5) * lng_ref[...] + lnb_ref[...]
    u = jnp.dot(xln, rtW_ref[...], preferred_element_type=jnp.float32)
    o_ref[...] = u + rtb_ref[...]


def _tc_middle(h0, h1, dinvb, g2, gb, preW, preb, lng, lnb, rtW, rtb, bm=1024):
    m = h0.shape[0]
    row64 = lambda i: (i, 0)
    fixed = lambda i: (0, 0)
    return pl.pallas_call(
        _mid_kernel,
        grid=(m // bm,),
        in_specs=[
            pl.BlockSpec((bm, H), row64), pl.BlockSpec((bm, H), row64),
            pl.BlockSpec((bm, H), row64), pl.BlockSpec((bm, H), row64),
            pl.BlockSpec((1, H), fixed),
            pl.BlockSpec((H, 128), fixed), pl.BlockSpec((1, 128), fixed),
            pl.BlockSpec((1, 128), fixed), pl.BlockSpec((1, 128), fixed),
            pl.BlockSpec((128, 128), fixed), pl.BlockSpec((1, 128), fixed),
        ],
        out_specs=pl.BlockSpec((bm, 128), row64),
        out_shape=jax.ShapeDtypeStruct((m, 128), jnp.float32),
    )(h0, h1, dinvb, g2, gb, preW, preb, lng, lnb, rtW, rtb)


@functools.partial(
    pl.kernel,
    out_type=jax.ShapeDtypeStruct((NW, NROW, 16), jnp.float32),
    mesh=_MESH, compiler_params=_SC_PARAMS,
    scratch_types=[
        pltpu.VMEM((NG, G), jnp.int32),
        pltpu.VMEM((NROW, 16), jnp.float32),
    ],
)
def _sc_deg(dst_hbm, deg_part, dstb, degl):
    w = _wid()
    pltpu.sync_copy(dst_hbm.at[w], dstb)

    @pl.loop(0, NROW)
    def _(j):
        degl[j, :] = jnp.zeros((16,), jnp.float32)

    onev = jnp.ones((16,), jnp.float32)

    @pl.loop(0, NG)
    def _(g):
        for i in range(G // 16):
            idx = dstb[g, pl.ds(i * 16, 16)]
            plsc.addupdate_scatter(degl, [idx >> 4, idx & 15], onev)

    pltpu.sync_copy(degl, deg_part.at[w])


NPW = N_PAD // NW
RPW = NROW // NW

@functools.partial(
    pl.kernel,
    out_type=[
        jax.ShapeDtypeStruct((NROW, 16), jnp.float32),
        jax.ShapeDtypeStruct((N_PAD, H), jnp.float32),
        jax.ShapeDtypeStruct((N_PAD, H), jnp.float32),
        jax.ShapeDtypeStruct((N_PAD, H), jnp.float32),
    ],
    mesh=_MESH, compiler_params=_SC_PARAMS,
    scratch_types=[
        pltpu.VMEM((RPW, 16), jnp.float32),
        pltpu.VMEM((RPW, 16), jnp.float32),
        pltpu.VMEM((RPW, 16), jnp.float32),
        pltpu.VMEM((NPW, H), jnp.float32),
        pltpu.VMEM((NPW, H), jnp.float32),
        pltpu.VMEM((NPW, H), jnp.float32),
    ],
)
def _sc_nodeprep(deg_part, hw_hbm, s0_out, g_out, g2_out, dinvb_out,
                 acc, tmp, dnv, hbuf, h2buf, h3buf):
    w = _wid()
    for j in range(RPW):
        acc[j, :] = jnp.zeros((16,), jnp.float32)

    @pl.loop(0, NW)
    def _(w2):
        pltpu.sync_copy(deg_part.at[w2, pl.ds(w * RPW, RPW)], tmp)
        for j in range(RPW):
            acc[j, :] = acc[j, :] + tmp[j, :]

    for j in range(RPW):
        dnv[j, :] = _rsqrt_nt(acc[j, :] + 1.0)
    pltpu.sync_copy(acc, s0_out.at[pl.ds(w * RPW, RPW)])

    pltpu.sync_copy(hw_hbm.at[pl.ds(w * NPW, NPW)], hbuf)

    @pl.loop(0, NPW // 16)
    def _(t):
        n16 = IOTA16() + t * 16
        dvv = dnv[t, :]
        for k in range(H):
            kv = jnp.full((16,), k, jnp.int32)
            row = plsc.load_gather(hbuf, [n16, kv])
            gg = row * dvv
            plsc.store_scatter(hbuf, [n16, kv], gg)
            plsc.store_scatter(h2buf, [n16, kv], gg * dvv)
            plsc.store_scatter(h3buf, [n16, kv], dvv)

    pltpu.sync_copy(hbuf, g_out.at[pl.ds(w * NPW, NPW)])
    pltpu.sync_copy(h2buf, g2_out.at[pl.ds(w * NPW, NPW)])
    pltpu.sync_copy(h3buf, dinvb_out.at[pl.ds(w * NPW, NPW)])


NPS = N_PAD // NS
ZCH = 160

def _make_dma_pass(W):
    @functools.partial(
        pl.kernel,
        out_type=jax.ShapeDtypeStruct((NC, N_PAD, W), jnp.float32),
        mesh=_MESH, compiler_params=_SC_PARAMS,
        scratch_types=[
            pltpu.VMEM((NG, G), jnp.int32),
            pltpu.VMEM((NG, G), jnp.int32),
            pltpu.VMEM((G, W), jnp.float32),
            pltpu.VMEM((G, W), jnp.float32),
            pltpu.VMEM((ZCH, W), jnp.float32),
            pltpu.VMEM_SHARED((N_PAD, W), jnp.float32),
            pltpu.SemaphoreType.DMA,
            pltpu.SemaphoreType.DMA,
        ],
    )
    def dma_pass(src_hbm, dst_hbm, tab_hbm, acc_part,
                 sb, db, rows0, rows1, zb, acc_sh, sem0, sem1):
        w = _wid()
        cid = lax.axis_index("c")
        sid = lax.axis_index("s")
        pltpu.sync_copy(src_hbm.at[w], sb)
        pltpu.sync_copy(dst_hbm.at[w], db)

        @pl.loop(0, ZCH)
        def _(j):
            for k in range(W // 16):
                zb[j, pl.ds(k * 16, 16)] = jnp.zeros((16,), jnp.float32)

        for z in range(NPS // ZCH):
            pltpu.sync_copy(zb, acc_sh.at[pl.ds(sid * NPS + z * ZCH, ZCH)])
        plsc.subcore_barrier()

        @pl.loop(0, NG // 2)
        def _(t):
            g0 = t * 2
            g1 = g0 + 1
            cp0 = pltpu.async_copy(tab_hbm.at[sb.at[g0]], rows0, sem0)
            cp1 = pltpu.async_copy(tab_hbm.at[sb.at[g1]], rows1, sem1)
            cp0.wait()
            pltpu.sync_copy(rows0, acc_sh.at[db.at[g0]], add=True)
            cp1.wait()
            pltpu.sync_copy(rows1, acc_sh.at[db.at[g1]], add=True)

        plsc.subcore_barrier()
        for z in range(NPS // ZCH):
            sl = pl.ds(sid * NPS + z * ZCH, ZCH)
            pltpu.sync_copy(acc_sh.at[sl], acc_part.at[cid, sl])

    return dma_pass


_sc_gcn = _make_dma_pass(H)
_sc_vpass0 = _make_dma_pass(CAP_PAD)


@functools.partial(
    pl.kernel,
    out_type=jax.ShapeDtypeStruct((N_PAD, CAP_PAD), jnp.float32),
    mesh=_MESH, compiler_params=_SC_PARAMS,
    scratch_types=[
        pltpu.VMEM((NPW, CAP_PAD), jnp.float32),
        pltpu.VMEM((NPW, CAP_PAD), jnp.float32),
        pltpu.VMEM((RPW, 16), jnp.float32),
        pltpu.VMEM((RPW, 16), jnp.float32),
    ],
)
def _sc_squash(v_part, s_a, s_b, v_out, va, vb, sa, sbuf):
    w = _wid()
    sl = pl.ds(w * NPW, NPW)
    rsl = pl.ds(w * RPW, RPW)
    pltpu.sync_copy(v_part.at[0, sl], va)
    pltpu.sync_copy(v_part.at[1, sl], vb)
    pltpu.sync_copy(s_a.at[rsl], sa)
    pltpu.sync_copy(s_b.at[rsl], sbuf)

    @pl.loop(0, NPW // 16)
    def _(t):
        n16 = IOTA16() + t * 16
        inv = 1.0 / (sa[t, :] + sbuf[t, :] + 1e-16)
        nsq = jnp.zeros((16,), jnp.float32)
        for k in range(CAP):
            kv = jnp.full((16,), k, jnp.int32)
            s = (plsc.load_gather(va, [n16, kv])
                 + plsc.load_gather(vb, [n16, kv])) * inv
            plsc.store_scatter(va, [n16, kv], s)
            nsq = nsq + s * s
        f = nsq / (1.0 + nsq) * _rsqrt_nt(nsq + 1e-9)
        for k in range(CAP):
            kv = jnp.full((16,), k, jnp.int32)
            s = plsc.load_gather(va, [n16, kv])
            plsc.store_scatter(va, [n16, kv], s * f)

    pltpu.sync_copy(va, v_out.at[sl])


def _make_sc_bpass(first):
    @functools.partial(
        pl.kernel,
        out_type=[
            jax.ShapeDtypeStruct((NW, NG, G), jnp.float32),
            jax.ShapeDtypeStruct((NC, NROW, 16), jnp.float32),
            jax.ShapeDtypeStruct((NC, N_PAD, CAP_PAD), jnp.float32),
        ],
        mesh=_MESH, compiler_params=_SC_PARAMS,
        scratch_types=[
            pltpu.VMEM((NG, G), jnp.int32),
            pltpu.VMEM((NG, G), jnp.int32),
            pltpu.VMEM((NG, G), jnp.float32),
            pltpu.VMEM((NROW, 16), jnp.float32),
            pltpu.VMEM((G, CAP_PAD), jnp.float32),
            pltpu.VMEM((G, CAP_PAD), jnp.float32),
            pltpu.VMEM((G, CAP_PAD), jnp.float32),
            pltpu.VMEM((G, CAP_PAD), jnp.float32),
            pltpu.VMEM((G, CAP_PAD), jnp.float32),
            pltpu.VMEM((G, CAP_PAD), jnp.float32),
            pltpu.VMEM((ZCH, CAP_PAD), jnp.float32),
            pltpu.VMEM((NROW // NS, 16), jnp.float32),
            pltpu.VMEM((NROW // NS, 16), jnp.float32),
            pltpu.VMEM_SHARED((NS, NROW, 16), jnp.float32),
            pltpu.VMEM_SHARED((N_PAD, CAP_PAD), jnp.float32),
            pltpu.SemaphoreType.DMA,
            pltpu.SemaphoreType.DMA,
            pltpu.SemaphoreType.DMA,
            pltpu.SemaphoreType.DMA,
        ],
    )
    def bpass(src_hbm, dst_hbm, u_hbm, v_hbm, b_in, b_out, s_part, vraw_part,
              sb, db, bb, sloc, ub0, vb0, ub1, vb1, cu0, cu1, zb, acc, tmp,
              s_sh, v_sh, sem0, sem1, sem2, sem3):
        w = _wid()
        cid = lax.axis_index("c")
        sid = lax.axis_index("s")
        pltpu.sync_copy(src_hbm.at[w], sb)
        pltpu.sync_copy(dst_hbm.at[w], db)
        if not first:
            pltpu.sync_copy(b_in.at[w], bb)

        @pl.loop(0, NROW)
        def _(j):
            sloc[j, :] = jnp.zeros((16,), jnp.float32)

        @pl.loop(0, ZCH)
        def _(j):
            for k in range(CAP_PAD // 16):
                zb[j, pl.ds(k * 16, 16)] = jnp.zeros((16,), jnp.float32)

        for z in range(NPS // ZCH):
            pltpu.sync_copy(zb, v_sh.at[pl.ds(sid * NPS + z * ZCH, ZCH)])

        @pl.loop(0, G)
        def _(j):
            for k in range(CAP_PAD // 16):
                cu0[j, pl.ds(k * 16, 16)] = jnp.zeros((16,), jnp.float32)
                cu1[j, pl.ds(k * 16, 16)] = jnp.zeros((16,), jnp.float32)
        plsc.subcore_barrier()

        def compute(g, ub, vb, cu):
            for i in range(G // 16):
                ev = IOTA16() + (i * 16)
                accv = jnp.zeros((16,), jnp.float32)
                for k in range(CAP):
                    kv = jnp.full((16,), k, jnp.int32)
                    uk = plsc.load_gather(ub, [ev, kv])
                    vk = plsc.load_gather(vb, [ev, kv])
                    accv = accv + uk * vk
                if first:
                    bnew = accv
                else:
                    bnew = bb[g, pl.ds(i * 16, 16)] + accv
                bb[g, pl.ds(i * 16, 16)] = bnew
                ex = jnp.exp(bnew)
                dv = db[g, pl.ds(i * 16, 16)]
                plsc.addupdate_scatter(sloc, [dv >> 4, dv & 15], ex)
                for k in range(CAP):
                    kv = jnp.full((16,), k, jnp.int32)
                    uk = plsc.load_gather(ub, [ev, kv])
                    plsc.store_scatter(cu, [ev, kv], uk * ex)

        @pl.loop(0, NG // 2)
        def _(t):
            g0 = t * 2
            g1 = g0 + 1
            du0 = pltpu.async_copy(u_hbm.at[sb.at[g0]], ub0, sem0)
            dv0 = pltpu.async_copy(v_hbm.at[db.at[g0]], vb0, sem1)
            du1 = pltpu.async_copy(u_hbm.at[sb.at[g1]], ub1, sem2)
            dv1 = pltpu.async_copy(v_hbm.at[db.at[g1]], vb1, sem3)
            du0.wait()
            dv0.wait()
            compute(g0, ub0, vb0, cu0)
            pltpu.sync_copy(cu0, v_sh.at[db.at[g0]], add=True)
            du1.wait()
            dv1.wait()
            compute(g1, ub1, vb1, cu1)
            pltpu.sync_copy(cu1, v_sh.at[db.at[g1]], add=True)

        pltpu.sync_copy(bb, b_out.at[w])
        pltpu.sync_copy(sloc, s_sh.at[sid])
        plsc.subcore_barrier()

        rps = NROW // NS
        for j in range(rps):
            acc[j, :] = jnp.zeros((16,), jnp.float32)

        @pl.loop(0, NS)
        def _(j):
            pltpu.sync_copy(s_sh.at[j, pl.ds(sid * rps, rps)], tmp)
            for r in range(rps):
                acc[r, :] = acc[r, :] + tmp[r, :]

        pltpu.sync_copy(acc, s_part.at[cid, pl.ds(sid * rps, rps)])
        for z in range(NPS // ZCH):
            sl = pl.ds(sid * NPS + z * ZCH, ZCH)
            pltpu.sync_copy(v_sh.at[sl], vraw_part.at[cid, sl])

    return bpass


_sc_bpass_first = _make_sc_bpass(True)
_sc_bpass_later = _make_sc_bpass(False)


def kernel(x, edge_index, gcn_W, gcn_b, pre_W, pre_b, ln_g, ln_b, rt_W, rt_b,
           cls_W, cls_b):
    src = edge_index[0]
    dst = edge_index[1]
    padi = jnp.full((E_PAD - E,), N_PAD - 1, dtype=src.dtype)
    srcp = jnp.concatenate([src, padi]).reshape(NW, NG, G).astype(jnp.int32)
    dstp = jnp.concatenate([dst, padi]).reshape(NW, NG, G).astype(jnp.int32)

    xp = jnp.pad(x, ((0, N_PAD - N), (0, 0)))
    hW = _tc_matmul(xp, gcn_W)

    deg_part = _sc_deg(dstp)
    s0, g_tab, g2_tab, dinvb = _sc_nodeprep(deg_part, hW)

    hsum_part = _sc_gcn(srcp, dstp, g_tab)

    preWp = jnp.pad(pre_W, ((0, 0), (0, 128 - CAP)))
    prebp = jnp.pad(pre_b, (0, 128 - CAP)).reshape(1, 128)
    lngp = jnp.pad(ln_g, (0, 128 - CAP)).reshape(1, 128)
    lnbp = jnp.pad(ln_b, (0, 128 - CAP)).reshape(1, 128)
    rtWp = jnp.pad(rt_W, ((0, 128 - CAP), (0, 128 - CAP)))
    rtbp = jnp.pad(rt_b, (0, 128 - CAP)).reshape(1, 128)
    gbp = jnp.broadcast_to(gcn_b.reshape(1, H), (1, H))

    u_full = _tc_middle(hsum_part[0], hsum_part[1], dinvb, g2_tab, gbp,
                        preWp, prebp, lngp, lnbp, rtWp, rtbp)
    u32 = u_full[:, :CAP_PAD]

    dummy_b = jnp.zeros((NW, NG, G), jnp.float32)
    dummy_s = jnp.zeros((NROW, 16), jnp.float32)

    v_part = _sc_vpass0(srcp, dstp, u32)
    v = _sc_squash(v_part, s0, dummy_s)
    b, s_part, v_part = _sc_bpass_first(srcp, dstp, u32, v, dummy_b)
    v = _sc_squash(v_part, s_part[0], s_part[1])
    b, s_part, v_part = _sc_bpass_later(srcp, dstp, u32, v, b)
    v = _sc_squash(v_part, s_part[0], s_part[1])

    clsWp = jnp.pad(cls_W, ((0, CAP_PAD - CAP), (0, 128 - C)))
    clsbp = jnp.pad(cls_b, (0, 128 - C)).reshape(1, 128)
    out = pl.pallas_call(
        _mm_kernel,
        grid=(N_PAD // 1024,),
        in_specs=[pl.BlockSpec((1024, CAP_PAD), lambda i: (i, 0)),
                  pl.BlockSpec((CAP_PAD, 128), lambda i: (0, 0))],
        out_specs=pl.BlockSpec((1024, 128), lambda i: (i, 0)),
        out_shape=jax.ShapeDtypeStruct((N_PAD, 128), jnp.float32),
    )(v, clsWp)
    out = out + clsbp
    return out[:N, :C]


# speedup vs baseline: 16.3649x; 1.2125x over previous
import functools
import jax
import jax.numpy as jnp
import numpy as np
from jax import lax
from jax.experimental import pallas as pl
from jax.experimental.pallas import tpu as pltpu
from jax.experimental.pallas import tpu_sc as plsc

N = 10000
E = 320000
F_IN = 128
H = 64
CAP = 18
C = 7
ITERS = 3

NC, NS, L = 2, 16, 16
NW = NC * NS
N_PAD = 10240
E_PAD = 327680
G = 128
NG = E_PAD // (NW * G)
NROW = N_PAD // 16
CAP_PAD = 32

_SC_PARAMS = pltpu.CompilerParams(
    use_tc_tiling_on_sc=False, needs_layout_passes=False)
_MESH = plsc.VectorSubcoreMesh(core_axis_name="c", subcore_axis_name="s")

IOTA16 = lambda: lax.iota(jnp.int32, 16)


def _rsqrt_nt(x):
    bits = lax.bitcast_convert_type(x, jnp.int32)
    magic = jnp.full(x.shape, 0x5F3759DF, jnp.int32)
    y = lax.bitcast_convert_type(magic - lax.shift_right_arithmetic(bits, 1),
                                 jnp.float32)
    for _ in range(4):
        y = y * (1.5 - 0.5 * x * y * y)
    return y


def _wid():
    return lax.axis_index("s") * NC + lax.axis_index("c")


def _mm_kernel(x_ref, w_ref, o_ref):
    o_ref[...] = jnp.dot(x_ref[...], w_ref[...],
                         preferred_element_type=jnp.float32)


def _tc_matmul(x, w, bm=1024):
    m, k = x.shape
    _, n = w.shape
    return pl.pallas_call(
        _mm_kernel,
        grid=(m // bm,),
        in_specs=[pl.BlockSpec((bm, k), lambda i: (i, 0)),
                  pl.BlockSpec((k, n), lambda i: (0, 0))],
        out_specs=pl.BlockSpec((bm, n), lambda i: (i, 0)),
        out_shape=jax.ShapeDtypeStruct((m, n), jnp.float32),
    )(x, w)


def _mid_kernel(h0_ref, h1_ref, dinvb_ref, g2_ref, gb_ref, preW_ref, preb_ref,
                lng_ref, lnb_ref, rtW_ref, rtb_ref, o_ref):
    hs = h0_ref[...] + h1_ref[...]
    h = jax.nn.relu(dinvb_ref[...] * hs + g2_ref[...] + gb_ref[...])
    hp = jnp.dot(h, preW_ref[...], preferred_element_type=jnp.float32)
    hp = hp + preb_ref[...]
    mu = jnp.sum(hp, axis=-1, keepdims=True) * (1.0 / CAP)
    m2 = jnp.sum(hp * hp, axis=-1, keepdims=True) * (1.0 / CAP)
    var = m2 - mu * mu
    xln = (hp - mu) * lax.rsqrt(var + 1e-5) * lng_ref[...] + lnb_ref[...]
    u = jnp.dot(xln, rtW_ref[...], preferred_element_type=jnp.float32)
    o_ref[...] = u + rtb_ref[...]


def _tc_middle(h0, h1, dinvb, g2, gb, preW, preb, lng, lnb, rtW, rtb, bm=1024):
    m = h0.shape[0]
    row64 = lambda i: (i, 0)
    fixed = lambda i: (0, 0)
    return pl.pallas_call(
        _mid_kernel,
        grid=(m // bm,),
        in_specs=[
            pl.BlockSpec((bm, H), row64), pl.BlockSpec((bm, H), row64),
            pl.BlockSpec((bm, H), row64), pl.BlockSpec((bm, H), row64),
            pl.BlockSpec((1, H), fixed),
            pl.BlockSpec((H, 128), fixed), pl.BlockSpec((1, 128), fixed),
            pl.BlockSpec((1, 128), fixed), pl.BlockSpec((1, 128), fixed),
            pl.BlockSpec((128, 128), fixed), pl.BlockSpec((1, 128), fixed),
        ],
        out_specs=pl.BlockSpec((bm, 128), row64),
        out_shape=jax.ShapeDtypeStruct((m, 128), jnp.float32),
    )(h0, h1, dinvb, g2, gb, preW, preb, lng, lnb, rtW, rtb)


@functools.partial(
    pl.kernel,
    out_type=jax.ShapeDtypeStruct((NW, NROW, 16), jnp.float32),
    mesh=_MESH, compiler_params=_SC_PARAMS,
    scratch_types=[
        pltpu.VMEM((NG, G), jnp.int32),
        pltpu.VMEM((NROW, 16), jnp.float32),
    ],
)
def _sc_deg(dst_hbm, deg_part, dstb, degl):
    w = _wid()
    pltpu.sync_copy(dst_hbm.at[w], dstb)

    @pl.loop(0, NROW)
    def _(j):
        degl[j, :] = jnp.zeros((16,), jnp.float32)

    onev = jnp.ones((16,), jnp.float32)

    @pl.loop(0, NG)
    def _(g):
        for i in range(G // 16):
            idx = dstb[g, pl.ds(i * 16, 16)]
            plsc.addupdate_scatter(degl, [idx >> 4, idx & 15], onev)

    pltpu.sync_copy(degl, deg_part.at[w])


NPW = N_PAD // NW
RPW = NROW // NW

@functools.partial(
    pl.kernel,
    out_type=[
        jax.ShapeDtypeStruct((NROW, 16), jnp.float32),
        jax.ShapeDtypeStruct((N_PAD, H), jnp.float32),
        jax.ShapeDtypeStruct((N_PAD, H), jnp.float32),
        jax.ShapeDtypeStruct((N_PAD, H), jnp.float32),
    ],
    mesh=_MESH, compiler_params=_SC_PARAMS,
    scratch_types=[
        pltpu.VMEM((RPW, 16), jnp.float32),
        pltpu.VMEM((RPW, 16), jnp.float32),
        pltpu.VMEM((RPW, 16), jnp.float32),
        pltpu.VMEM((NPW, H), jnp.float32),
        pltpu.VMEM((NPW, H), jnp.float32),
        pltpu.VMEM((NPW, H), jnp.float32),
    ],
)
def _sc_nodeprep(deg_part, hw_hbm, s0_out, g_out, g2_out, dinvb_out,
                 acc, tmp, dnv, hbuf, h2buf, h3buf):
    w = _wid()
    for j in range(RPW):
        acc[j, :] = jnp.zeros((16,), jnp.float32)

    @pl.loop(0, NW)
    def _(w2):
        pltpu.sync_copy(deg_part.at[w2, pl.ds(w * RPW, RPW)], tmp)
        for j in range(RPW):
            acc[j, :] = acc[j, :] + tmp[j, :]

    for j in range(RPW):
        dnv[j, :] = _rsqrt_nt(acc[j, :] + 1.0)
    pltpu.sync_copy(acc, s0_out.at[pl.ds(w * RPW, RPW)])

    pltpu.sync_copy(hw_hbm.at[pl.ds(w * NPW, NPW)], hbuf)

    @pl.loop(0, NPW // 16)
    def _(t):
        n16 = IOTA16() + t * 16
        dvv = dnv[t, :]
        for k in range(H):
            kv = jnp.full((16,), k, jnp.int32)
            row = plsc.load_gather(hbuf, [n16, kv])
            gg = row * dvv
            plsc.store_scatter(hbuf, [n16, kv], gg)
            plsc.store_scatter(h2buf, [n16, kv], gg * dvv)
            plsc.store_scatter(h3buf, [n16, kv], dvv)

    pltpu.sync_copy(hbuf, g_out.at[pl.ds(w * NPW, NPW)])
    pltpu.sync_copy(h2buf, g2_out.at[pl.ds(w * NPW, NPW)])
    pltpu.sync_copy(h3buf, dinvb_out.at[pl.ds(w * NPW, NPW)])


NPS = N_PAD // NS
ZCH = 160

def _make_dma_pass(W):
    @functools.partial(
        pl.kernel,
        out_type=jax.ShapeDtypeStruct((NC, N_PAD, W), jnp.float32),
        mesh=_MESH, compiler_params=_SC_PARAMS,
        scratch_types=[
            pltpu.VMEM((NG, G), jnp.int32),
            pltpu.VMEM((NG, G), jnp.int32),
            pltpu.VMEM((G, W), jnp.float32),
            pltpu.VMEM((G, W), jnp.float32),
            pltpu.VMEM((ZCH, W), jnp.float32),
            pltpu.VMEM_SHARED((N_PAD, W), jnp.float32),
            pltpu.SemaphoreType.DMA,
            pltpu.SemaphoreType.DMA,
            pltpu.SemaphoreType.DMA,
            pltpu.SemaphoreType.DMA,
        ],
    )
    def dma_pass(src_hbm, dst_hbm, tab_hbm, acc_part,
                 sb, db, rows0, rows1, zb, acc_sh, sem0, sem1, sem2, sem3):
        w = _wid()
        cid = lax.axis_index("c")
        sid = lax.axis_index("s")
        pltpu.sync_copy(src_hbm.at[w], sb)
        pltpu.sync_copy(dst_hbm.at[w], db)

        @pl.loop(0, ZCH)
        def _(j):
            for k in range(W // 16):
                zb[j, pl.ds(k * 16, 16)] = jnp.zeros((16,), jnp.float32)

        for z in range(NPS // ZCH):
            pltpu.sync_copy(zb, acc_sh.at[pl.ds(sid * NPS + z * ZCH, ZCH)])
        plsc.subcore_barrier()

        @pl.loop(0, NG // 2)
        def _(t):
            g0 = t * 2
            g1 = g0 + 1
            cp0 = pltpu.async_copy(tab_hbm.at[sb.at[g0]], rows0, sem0)
            cp1 = pltpu.async_copy(tab_hbm.at[sb.at[g1]], rows1, sem1)
            cp0.wait()
            sc0 = pltpu.async_copy(rows0, acc_sh.at[db.at[g0]], sem2, add=True)
            cp1.wait()
            sc1 = pltpu.async_copy(rows1, acc_sh.at[db.at[g1]], sem3, add=True)
            sc0.wait()
            sc1.wait()

        plsc.subcore_barrier()
        for z in range(NPS // ZCH):
            sl = pl.ds(sid * NPS + z * ZCH, ZCH)
            pltpu.sync_copy(acc_sh.at[sl], acc_part.at[cid, sl])

    return dma_pass


_sc_gcn = _make_dma_pass(H)
_sc_vpass0 = _make_dma_pass(CAP_PAD)


@functools.partial(
    pl.kernel,
    out_type=jax.ShapeDtypeStruct((N_PAD, CAP_PAD), jnp.float32),
    mesh=_MESH, compiler_params=_SC_PARAMS,
    scratch_types=[
        pltpu.VMEM((NPW, CAP_PAD), jnp.float32),
        pltpu.VMEM((NPW, CAP_PAD), jnp.float32),
        pltpu.VMEM((RPW, 16), jnp.float32),
        pltpu.VMEM((RPW, 16), jnp.float32),
    ],
)
def _sc_squash(v_part, s_a, s_b, v_out, va, vb, sa, sbuf):
    w = _wid()
    sl = pl.ds(w * NPW, NPW)
    rsl = pl.ds(w * RPW, RPW)
    pltpu.sync_copy(v_part.at[0, sl], va)
    pltpu.sync_copy(v_part.at[1, sl], vb)
    pltpu.sync_copy(s_a.at[rsl], sa)
    pltpu.sync_copy(s_b.at[rsl], sbuf)

    @pl.loop(0, NPW // 16)
    def _(t):
        n16 = IOTA16() + t * 16
        inv = 1.0 / (sa[t, :] + sbuf[t, :] + 1e-16)
        nsq = jnp.zeros((16,), jnp.float32)
        for k in range(CAP):
            kv = jnp.full((16,), k, jnp.int32)
            s = (plsc.load_gather(va, [n16, kv])
                 + plsc.load_gather(vb, [n16, kv])) * inv
            plsc.store_scatter(va, [n16, kv], s)
            nsq = nsq + s * s
        f = nsq / (1.0 + nsq) * _rsqrt_nt(nsq + 1e-9)
        for k in range(CAP):
            kv = jnp.full((16,), k, jnp.int32)
            s = plsc.load_gather(va, [n16, kv])
            plsc.store_scatter(va, [n16, kv], s * f)

    pltpu.sync_copy(va, v_out.at[sl])


def _make_sc_bpass(first):
    @functools.partial(
        pl.kernel,
        out_type=[
            jax.ShapeDtypeStruct((NW, NG, G), jnp.float32),
            jax.ShapeDtypeStruct((NC, NROW, 16), jnp.float32),
            jax.ShapeDtypeStruct((NC, N_PAD, CAP_PAD), jnp.float32),
        ],
        mesh=_MESH, compiler_params=_SC_PARAMS,
        scratch_types=[
            pltpu.VMEM((NG, G), jnp.int32),
            pltpu.VMEM((NG, G), jnp.int32),
            pltpu.VMEM((NG, G), jnp.float32),
            pltpu.VMEM((NROW, 16), jnp.float32),
            pltpu.VMEM((G, CAP_PAD), jnp.float32),
            pltpu.VMEM((G, CAP_PAD), jnp.float32),
            pltpu.VMEM((G, CAP_PAD), jnp.float32),
            pltpu.VMEM((G, CAP_PAD), jnp.float32),
            pltpu.VMEM((G, CAP_PAD), jnp.float32),
            pltpu.VMEM((G, CAP_PAD), jnp.float32),
            pltpu.VMEM((ZCH, CAP_PAD), jnp.float32),
            pltpu.VMEM((NROW // NS, 16), jnp.float32),
            pltpu.VMEM((NROW // NS, 16), jnp.float32),
            pltpu.VMEM_SHARED((NS, NROW, 16), jnp.float32),
            pltpu.VMEM_SHARED((N_PAD, CAP_PAD), jnp.float32),
            pltpu.SemaphoreType.DMA,
            pltpu.SemaphoreType.DMA,
            pltpu.SemaphoreType.DMA,
            pltpu.SemaphoreType.DMA,
            pltpu.SemaphoreType.DMA,
            pltpu.SemaphoreType.DMA,
        ],
    )
    def bpass(src_hbm, dst_hbm, u_hbm, v_hbm, b_in, b_out, s_part, vraw_part,
              sb, db, bb, sloc, ub0, vb0, ub1, vb1, cu0, cu1, zb, acc, tmp,
              s_sh, v_sh, sem0, sem1, sem2, sem3, sem4, sem5):
        w = _wid()
        cid = lax.axis_index("c")
        sid = lax.axis_index("s")
        pltpu.sync_copy(src_hbm.at[w], sb)
        pltpu.sync_copy(dst_hbm.at[w], db)
        if not first:
            pltpu.sync_copy(b_in.at[w], bb)

        @pl.loop(0, NROW)
        def _(j):
            sloc[j, :] = jnp.zeros((16,), jnp.float32)

        @pl.loop(0, ZCH)
        def _(j):
            for k in range(CAP_PAD // 16):
                zb[j, pl.ds(k * 16, 16)] = jnp.zeros((16,), jnp.float32)

        for z in range(NPS // ZCH):
            pltpu.sync_copy(zb, v_sh.at[pl.ds(sid * NPS + z * ZCH, ZCH)])

        @pl.loop(0, G)
        def _(j):
            for k in range(CAP_PAD // 16):
                cu0[j, pl.ds(k * 16, 16)] = jnp.zeros((16,), jnp.float32)
                cu1[j, pl.ds(k * 16, 16)] = jnp.zeros((16,), jnp.float32)
        plsc.subcore_barrier()

        def compute(g, ub, vb, cu):
            for i in range(G // 16):
                ev = IOTA16() + (i * 16)
                accs = [jnp.zeros((16,), jnp.float32) for _ in range(4)]
                uks = []
                for k in range(CAP):
                    kv = jnp.full((16,), k, jnp.int32)
                    uk = plsc.load_gather(ub, [ev, kv])
                    vk = plsc.load_gather(vb, [ev, kv])
                    uks.append(uk)
                    accs[k % 4] = accs[k % 4] + uk * vk
                accv = (accs[0] + accs[1]) + (accs[2] + accs[3])
                if first:
                    bnew = accv
                else:
                    bnew = bb[g, pl.ds(i * 16, 16)] + accv
                bb[g, pl.ds(i * 16, 16)] = bnew
                ex = jnp.exp(bnew)
                dv = db[g, pl.ds(i * 16, 16)]
                plsc.addupdate_scatter(sloc, [dv >> 4, dv & 15], ex)
                for k in range(CAP):
                    kv = jnp.full((16,), k, jnp.int32)
                    plsc.store_scatter(cu, [ev, kv], uks[k] * ex)

        @pl.loop(0, NG // 2)
        def _(t):
            g0 = t * 2
            g1 = g0 + 1
            du0 = pltpu.async_copy(u_hbm.at[sb.at[g0]], ub0, sem0)
            dv0 = pltpu.async_copy(v_hbm.at[db.at[g0]], vb0, sem1)
            du1 = pltpu.async_copy(u_hbm.at[sb.at[g1]], ub1, sem2)
            dv1 = pltpu.async_copy(v_hbm.at[db.at[g1]], vb1, sem3)
            du0.wait()
            dv0.wait()
            compute(g0, ub0, vb0, cu0)
            sc0 = pltpu.async_copy(cu0, v_sh.at[db.at[g0]], sem4, add=True)
            du1.wait()
            dv1.wait()
            compute(g1, ub1, vb1, cu1)
            sc1 = pltpu.async_copy(cu1, v_sh.at[db.at[g1]], sem5, add=True)
            sc0.wait()
            sc1.wait()

        pltpu.sync_copy(bb, b_out.at[w])
        pltpu.sync_copy(sloc, s_sh.at[sid])
        plsc.subcore_barrier()

        rps = NROW // NS
        for j in range(rps):
            acc[j, :] = jnp.zeros((16,), jnp.float32)

        @pl.loop(0, NS)
        def _(j):
            pltpu.sync_copy(s_sh.at[j, pl.ds(sid * rps, rps)], tmp)
            for r in range(rps):
                acc[r, :] = acc[r, :] + tmp[r, :]

        pltpu.sync_copy(acc, s_part.at[cid, pl.ds(sid * rps, rps)])
        for z in range(NPS // ZCH):
            sl = pl.ds(sid * NPS + z * ZCH, ZCH)
            pltpu.sync_copy(v_sh.at[sl], vraw_part.at[cid, sl])

    return bpass


_sc_bpass_first = _make_sc_bpass(True)
_sc_bpass_later = _make_sc_bpass(False)


def kernel(x, edge_index, gcn_W, gcn_b, pre_W, pre_b, ln_g, ln_b, rt_W, rt_b,
           cls_W, cls_b):
    src = edge_index[0]
    dst = edge_index[1]
    padi = jnp.full((E_PAD - E,), N_PAD - 1, dtype=src.dtype)
    srcp = jnp.concatenate([src, padi]).reshape(NW, NG, G).astype(jnp.int32)
    dstp = jnp.concatenate([dst, padi]).reshape(NW, NG, G).astype(jnp.int32)

    xp = jnp.pad(x, ((0, N_PAD - N), (0, 0)))
    hW = _tc_matmul(xp, gcn_W)

    deg_part = _sc_deg(dstp)
    s0, g_tab, g2_tab, dinvb = _sc_nodeprep(deg_part, hW)

    hsum_part = _sc_gcn(srcp, dstp, g_tab)

    preWp = jnp.pad(pre_W, ((0, 0), (0, 128 - CAP)))
    prebp = jnp.pad(pre_b, (0, 128 - CAP)).reshape(1, 128)
    lngp = jnp.pad(ln_g, (0, 128 - CAP)).reshape(1, 128)
    lnbp = jnp.pad(ln_b, (0, 128 - CAP)).reshape(1, 128)
    rtWp = jnp.pad(rt_W, ((0, 128 - CAP), (0, 128 - CAP)))
    rtbp = jnp.pad(rt_b, (0, 128 - CAP)).reshape(1, 128)
    gbp = jnp.broadcast_to(gcn_b.reshape(1, H), (1, H))

    u_full = _tc_middle(hsum_part[0], hsum_part[1], dinvb, g2_tab, gbp,
                        preWp, prebp, lngp, lnbp, rtWp, rtbp)
    u32 = u_full[:, :CAP_PAD]

    dummy_b = jnp.zeros((NW, NG, G), jnp.float32)
    dummy_s = jnp.zeros((NROW, 16), jnp.float32)

    v_part = _sc_vpass0(srcp, dstp, u32)
    v = _sc_squash(v_part, s0, dummy_s)
    b, s_part, v_part = _sc_bpass_first(srcp, dstp, u32, v, dummy_b)
    v = _sc_squash(v_part, s_part[0], s_part[1])
    b, s_part, v_part = _sc_bpass_later(srcp, dstp, u32, v, b)
    v = _sc_squash(v_part, s_part[0], s_part[1])

    clsWp = jnp.pad(cls_W, ((0, CAP_PAD - CAP), (0, 128 - C)))
    clsbp = jnp.pad(cls_b, (0, 128 - C)).reshape(1, 128)
    out = pl.pallas_call(
        _mm_kernel,
        grid=(N_PAD // 1024,),
        in_specs=[pl.BlockSpec((1024, CAP_PAD), lambda i: (i, 0)),
                  pl.BlockSpec((CAP_PAD, 128), lambda i: (0, 0))],
        out_specs=pl.BlockSpec((1024, 128), lambda i: (i, 0)),
        out_shape=jax.ShapeDtypeStruct((N_PAD, 128), jnp.float32),
    )(v, clsWp)
    out = out + clsbp
    return out[:N, :C]


# speedup vs baseline: 16.5634x; 1.0121x over previous
import functools
import jax
import jax.numpy as jnp
import numpy as np
from jax import lax
from jax.experimental import pallas as pl
from jax.experimental.pallas import tpu as pltpu
from jax.experimental.pallas import tpu_sc as plsc

N = 10000
E = 320000
F_IN = 128
H = 64
CAP = 18
C = 7
ITERS = 3

NC, NS, L = 2, 16, 16
NW = NC * NS
N_PAD = 10240
E_PAD = 327680
G = 128
NG = E_PAD // (NW * G)
NROW = N_PAD // 16
CAP_PAD = 32

_SC_PARAMS = pltpu.CompilerParams(
    use_tc_tiling_on_sc=False, needs_layout_passes=False)
_MESH = plsc.VectorSubcoreMesh(core_axis_name="c", subcore_axis_name="s")

IOTA16 = lambda: lax.iota(jnp.int32, 16)


def _rsqrt_nt(x):
    bits = lax.bitcast_convert_type(x, jnp.int32)
    magic = jnp.full(x.shape, 0x5F3759DF, jnp.int32)
    y = lax.bitcast_convert_type(magic - lax.shift_right_arithmetic(bits, 1),
                                 jnp.float32)
    for _ in range(4):
        y = y * (1.5 - 0.5 * x * y * y)
    return y


def _wid():
    return lax.axis_index("s") * NC + lax.axis_index("c")


def _mm_kernel(x_ref, w_ref, o_ref):
    o_ref[...] = jnp.dot(x_ref[...], w_ref[...],
                         preferred_element_type=jnp.float32)


def _tc_matmul(x, w, bm=1024):
    m, k = x.shape
    _, n = w.shape
    return pl.pallas_call(
        _mm_kernel,
        grid=(m // bm,),
        in_specs=[pl.BlockSpec((bm, k), lambda i: (i, 0)),
                  pl.BlockSpec((k, n), lambda i: (0, 0))],
        out_specs=pl.BlockSpec((bm, n), lambda i: (i, 0)),
        out_shape=jax.ShapeDtypeStruct((m, n), jnp.float32),
    )(x, w)


def _mid_kernel(h0_ref, h1_ref, dinvb_ref, g2_ref, gb_ref, preW_ref, preb_ref,
                lng_ref, lnb_ref, rtW_ref, rtb_ref, o_ref):
    hs = h0_ref[...] + h1_ref[...]
    h = jax.nn.relu(dinvb_ref[...] * hs + g2_ref[...] + gb_ref[...])
    hp = jnp.dot(h, preW_ref[...], preferred_element_type=jnp.float32)
    hp = hp + preb_ref[...]
    mu = jnp.sum(hp, axis=-1, keepdims=True) * (1.0 / CAP)
    m2 = jnp.sum(hp * hp, axis=-1, keepdims=True) * (1.0 / CAP)
    var = m2 - mu * mu
    xln = (hp - mu) * lax.rsqrt(var + 1e-5) * lng_ref[...] + lnb_ref[...]
    u = jnp.dot(xln, rtW_ref[...], preferred_element_type=jnp.float32)
    o_ref[...] = u + rtb_ref[...]


def _tc_middle(h0, h1, dinvb, g2, gb, preW, preb, lng, lnb, rtW, rtb, bm=1024):
    m = h0.shape[0]
    row64 = lambda i: (i, 0)
    fixed = lambda i: (0, 0)
    return pl.pallas_call(
        _mid_kernel,
        grid=(m // bm,),
        in_specs=[
            pl.BlockSpec((bm, H), row64), pl.BlockSpec((bm, H), row64),
            pl.BlockSpec((bm, H), row64), pl.BlockSpec((bm, H), row64),
            pl.BlockSpec((1, H), fixed),
            pl.BlockSpec((H, 128), fixed), pl.BlockSpec((1, 128), fixed),
            pl.BlockSpec((1, 128), fixed), pl.BlockSpec((1, 128), fixed),
            pl.BlockSpec((128, 128), fixed), pl.BlockSpec((1, 128), fixed),
        ],
        out_specs=pl.BlockSpec((bm, 128), row64),
        out_shape=jax.ShapeDtypeStruct((m, 128), jnp.float32),
    )(h0, h1, dinvb, g2, gb, preW, preb, lng, lnb, rtW, rtb)


@functools.partial(
    pl.kernel,
    out_type=jax.ShapeDtypeStruct((NW, NROW, 16), jnp.float32),
    mesh=_MESH, compiler_params=_SC_PARAMS,
    scratch_types=[
        pltpu.VMEM((NG, G), jnp.int32),
        pltpu.VMEM((NROW, 16), jnp.float32),
    ],
)
def _sc_deg(dst_hbm, deg_part, dstb, degl):
    w = _wid()
    pltpu.sync_copy(dst_hbm.at[w], dstb)

    @pl.loop(0, NROW)
    def _(j):
        degl[j, :] = jnp.zeros((16,), jnp.float32)

    onev = jnp.ones((16,), jnp.float32)

    @pl.loop(0, NG)
    def _(g):
        for i in range(G // 16):
            idx = dstb[g, pl.ds(i * 16, 16)]
            plsc.addupdate_scatter(degl, [idx >> 4, idx & 15], onev)

    pltpu.sync_copy(degl, deg_part.at[w])


NPW = N_PAD // NW
RPW = NROW // NW

@functools.partial(
    pl.kernel,
    out_type=[
        jax.ShapeDtypeStruct((NROW, 16), jnp.float32),
        jax.ShapeDtypeStruct((N_PAD, H), jnp.float32),
        jax.ShapeDtypeStruct((N_PAD, H), jnp.float32),
        jax.ShapeDtypeStruct((N_PAD, H), jnp.float32),
    ],
    mesh=_MESH, compiler_params=_SC_PARAMS,
    scratch_types=[
        pltpu.VMEM((RPW, 16), jnp.float32),
        pltpu.VMEM((RPW, 16), jnp.float32),
        pltpu.VMEM((RPW, 16), jnp.float32),
        pltpu.VMEM((NPW, H), jnp.float32),
        pltpu.VMEM((NPW, H), jnp.float32),
        pltpu.VMEM((NPW, H), jnp.float32),
    ],
)
def _sc_nodeprep(deg_part, hw_hbm, s0_out, g_out, g2_out, dinvb_out,
                 acc, tmp, dnv, hbuf, h2buf, h3buf):
    w = _wid()
    for j in range(RPW):
        acc[j, :] = jnp.zeros((16,), jnp.float32)

    @pl.loop(0, NW)
    def _(w2):
        pltpu.sync_copy(deg_part.at[w2, pl.ds(w * RPW, RPW)], tmp)
        for j in range(RPW):
            acc[j, :] = acc[j, :] + tmp[j, :]

    for j in range(RPW):
        dnv[j, :] = _rsqrt_nt(acc[j, :] + 1.0)
    pltpu.sync_copy(acc, s0_out.at[pl.ds(w * RPW, RPW)])

    pltpu.sync_copy(hw_hbm.at[pl.ds(w * NPW, NPW)], hbuf)

    @pl.loop(0, NPW // 16)
    def _(t):
        n16 = IOTA16() + t * 16
        dvv = dnv[t, :]
        for k in range(H):
            kv = jnp.full((16,), k, jnp.int32)
            row = plsc.load_gather(hbuf, [n16, kv])
            gg = row * dvv
            plsc.store_scatter(hbuf, [n16, kv], gg)
            plsc.store_scatter(h2buf, [n16, kv], gg * dvv)
            plsc.store_scatter(h3buf, [n16, kv], dvv)

    pltpu.sync_copy(hbuf, g_out.at[pl.ds(w * NPW, NPW)])
    pltpu.sync_copy(h2buf, g2_out.at[pl.ds(w * NPW, NPW)])
    pltpu.sync_copy(h3buf, dinvb_out.at[pl.ds(w * NPW, NPW)])


NPS = N_PAD // NS
ZCH = 160

def _make_dma_pass(W):
    @functools.partial(
        pl.kernel,
        out_type=jax.ShapeDtypeStruct((NC, N_PAD, W), jnp.float32),
        mesh=_MESH, compiler_params=_SC_PARAMS,
        scratch_types=[
            pltpu.VMEM((NG, G), jnp.int32),
            pltpu.VMEM((NG, G), jnp.int32),
            pltpu.VMEM((G, W), jnp.float32),
            pltpu.VMEM((G, W), jnp.float32),
            pltpu.VMEM((G, W), jnp.float32),
            pltpu.VMEM((G, W), jnp.float32),
            pltpu.VMEM((ZCH, W), jnp.float32),
            pltpu.VMEM_SHARED((N_PAD, W), jnp.float32),
        ] + [pltpu.SemaphoreType.DMA] * 8,
    )
    def dma_pass(src_hbm, dst_hbm, tab_hbm, acc_part,
                 sb, db, rows0, rows1, rows2, rows3, zb, acc_sh,
                 gs0, gs1, gs2, gs3, ss0, ss1, ss2, ss3):
        w = _wid()
        cid = lax.axis_index("c")
        sid = lax.axis_index("s")
        pltpu.sync_copy(src_hbm.at[w], sb)
        pltpu.sync_copy(dst_hbm.at[w], db)

        @pl.loop(0, ZCH)
        def _(j):
            for k in range(W // 16):
                zb[j, pl.ds(k * 16, 16)] = jnp.zeros((16,), jnp.float32)

        for z in range(NPS // ZCH):
            pltpu.sync_copy(zb, acc_sh.at[pl.ds(sid * NPS + z * ZCH, ZCH)])
        plsc.subcore_barrier()

        rows = [rows0, rows1, rows2, rows3]
        gsems = [gs0, gs1, gs2, gs3]
        ssems = [ss0, ss1, ss2, ss3]

        @pl.loop(0, NG // 4)
        def _(t):
            gbase = t * 4
            cps = [pltpu.async_copy(tab_hbm.at[sb.at[gbase + q]], rows[q],
                                    gsems[q]) for q in range(4)]
            scs = []
            for q in range(4):
                cps[q].wait()
                scs.append(pltpu.async_copy(rows[q], acc_sh.at[db.at[gbase + q]],
                                            ssems[q], add=True))
            for q in range(4):
                scs[q].wait()

        plsc.subcore_barrier()
        for z in range(NPS // ZCH):
            sl = pl.ds(sid * NPS + z * ZCH, ZCH)
            pltpu.sync_copy(acc_sh.at[sl], acc_part.at[cid, sl])

    return dma_pass


_sc_gcn = _make_dma_pass(H)
_sc_vpass0 = _make_dma_pass(CAP_PAD)


@functools.partial(
    pl.kernel,
    out_type=jax.ShapeDtypeStruct((N_PAD, CAP_PAD), jnp.float32),
    mesh=_MESH, compiler_params=_SC_PARAMS,
    scratch_types=[
        pltpu.VMEM((NPW, CAP_PAD), jnp.float32),
        pltpu.VMEM((NPW, CAP_PAD), jnp.float32),
        pltpu.VMEM((RPW, 16), jnp.float32),
        pltpu.VMEM((RPW, 16), jnp.float32),
    ],
)
def _sc_squash(v_part, s_a, s_b, v_out, va, vb, sa, sbuf):
    w = _wid()
    sl = pl.ds(w * NPW, NPW)
    rsl = pl.ds(w * RPW, RPW)
    pltpu.sync_copy(v_part.at[0, sl], va)
    pltpu.sync_copy(v_part.at[1, sl], vb)
    pltpu.sync_copy(s_a.at[rsl], sa)
    pltpu.sync_copy(s_b.at[rsl], sbuf)

    @pl.loop(0, NPW // 16)
    def _(t):
        n16 = IOTA16() + t * 16
        inv = 1.0 / (sa[t, :] + sbuf[t, :] + 1e-16)
        nsq = jnp.zeros((16,), jnp.float32)
        for k in range(CAP):
            kv = jnp.full((16,), k, jnp.int32)
            s = (plsc.load_gather(va, [n16, kv])
                 + plsc.load_gather(vb, [n16, kv])) * inv
            plsc.store_scatter(va, [n16, kv], s)
            nsq = nsq + s * s
        f = nsq / (1.0 + nsq) * _rsqrt_nt(nsq + 1e-9)
        for k in range(CAP):
            kv = jnp.full((16,), k, jnp.int32)
            s = plsc.load_gather(va, [n16, kv])
            plsc.store_scatter(va, [n16, kv], s * f)

    pltpu.sync_copy(va, v_out.at[sl])


def _make_sc_bpass(first):
    @functools.partial(
        pl.kernel,
        out_type=[
            jax.ShapeDtypeStruct((NW, NG, G), jnp.float32),
            jax.ShapeDtypeStruct((NC, NROW, 16), jnp.float32),
            jax.ShapeDtypeStruct((NC, N_PAD, CAP_PAD), jnp.float32),
        ],
        mesh=_MESH, compiler_params=_SC_PARAMS,
        scratch_types=[
            pltpu.VMEM((NG, G), jnp.int32),
            pltpu.VMEM((NG, G), jnp.int32),
            pltpu.VMEM((NG, G), jnp.float32),
            pltpu.VMEM((NROW, 16), jnp.float32),
            pltpu.VMEM((G, CAP_PAD), jnp.float32),
            pltpu.VMEM((G, CAP_PAD), jnp.float32),
            pltpu.VMEM((G, CAP_PAD), jnp.float32),
            pltpu.VMEM((G, CAP_PAD), jnp.float32),
            pltpu.VMEM((G, CAP_PAD), jnp.float32),
            pltpu.VMEM((G, CAP_PAD), jnp.float32),
            pltpu.VMEM((G, CAP_PAD), jnp.float32),
            pltpu.VMEM((G, CAP_PAD), jnp.float32),
            pltpu.VMEM((G, CAP_PAD), jnp.float32),
            pltpu.VMEM((G, CAP_PAD), jnp.float32),
            pltpu.VMEM((G, CAP_PAD), jnp.float32),
            pltpu.VMEM((G, CAP_PAD), jnp.float32),
            pltpu.VMEM((ZCH, CAP_PAD), jnp.float32),
            pltpu.VMEM((NROW // NS, 16), jnp.float32),
            pltpu.VMEM((NROW // NS, 16), jnp.float32),
            pltpu.VMEM_SHARED((NS, NROW, 16), jnp.float32),
            pltpu.VMEM_SHARED((N_PAD, CAP_PAD), jnp.float32),
        ] + [pltpu.SemaphoreType.DMA] * 12,
    )
    def bpass(src_hbm, dst_hbm, u_hbm, v_hbm, b_in, b_out, s_part, vraw_part,
              sb, db, bb, sloc, ub0, vb0, ub1, vb1, ub2, vb2, ub3, vb3,
              cu0, cu1, cu2, cu3, zb, acc, tmp,
              s_sh, v_sh, us0, us1, us2, us3, vs0, vs1, vs2, vs3,
              cs0, cs1, cs2, cs3):
        w = _wid()
        cid = lax.axis_index("c")
        sid = lax.axis_index("s")
        pltpu.sync_copy(src_hbm.at[w], sb)
        pltpu.sync_copy(dst_hbm.at[w], db)
        if not first:
            pltpu.sync_copy(b_in.at[w], bb)

        @pl.loop(0, NROW)
        def _(j):
            sloc[j, :] = jnp.zeros((16,), jnp.float32)

        @pl.loop(0, ZCH)
        def _(j):
            for k in range(CAP_PAD // 16):
                zb[j, pl.ds(k * 16, 16)] = jnp.zeros((16,), jnp.float32)

        for z in range(NPS // ZCH):
            pltpu.sync_copy(zb, v_sh.at[pl.ds(sid * NPS + z * ZCH, ZCH)])

        @pl.loop(0, G)
        def _(j):
            for k in range(CAP_PAD // 16):
                cu0[j, pl.ds(k * 16, 16)] = jnp.zeros((16,), jnp.float32)
                cu1[j, pl.ds(k * 16, 16)] = jnp.zeros((16,), jnp.float32)
                cu2[j, pl.ds(k * 16, 16)] = jnp.zeros((16,), jnp.float32)
                cu3[j, pl.ds(k * 16, 16)] = jnp.zeros((16,), jnp.float32)
        plsc.subcore_barrier()

        def compute(g, ub, vb, cu):
            for i in range(G // 16):
                ev = IOTA16() + (i * 16)
                accs = [jnp.zeros((16,), jnp.float32) for _ in range(4)]
                uks = []
                for k in range(CAP):
                    kv = jnp.full((16,), k, jnp.int32)
                    uk = plsc.load_gather(ub, [ev, kv])
                    vk = plsc.load_gather(vb, [ev, kv])
                    uks.append(uk)
                    accs[k % 4] = accs[k % 4] + uk * vk
                accv = (accs[0] + accs[1]) + (accs[2] + accs[3])
                if first:
                    bnew = accv
                else:
                    bnew = bb[g, pl.ds(i * 16, 16)] + accv
                bb[g, pl.ds(i * 16, 16)] = bnew
                ex = jnp.exp(bnew)
                dv = db[g, pl.ds(i * 16, 16)]
                plsc.addupdate_scatter(sloc, [dv >> 4, dv & 15], ex)
                for k in range(CAP):
                    kv = jnp.full((16,), k, jnp.int32)
                    plsc.store_scatter(cu, [ev, kv], uks[k] * ex)

        ubs = [ub0, ub1, ub2, ub3]
        vbs = [vb0, vb1, vb2, vb3]
        cus = [cu0, cu1, cu2, cu3]
        usem = [us0, us1, us2, us3]
        vsem = [vs0, vs1, vs2, vs3]
        csem = [cs0, cs1, cs2, cs3]

        @pl.loop(0, NG // 4)
        def _(t):
            gbase = t * 4
            dus = [pltpu.async_copy(u_hbm.at[sb.at[gbase + q]], ubs[q], usem[q])
                   for q in range(4)]
            dvs = [pltpu.async_copy(v_hbm.at[db.at[gbase + q]], vbs[q], vsem[q])
                   for q in range(4)]
            scs = []
            for q in range(4):
                dus[q].wait()
                dvs[q].wait()
                compute(gbase + q, ubs[q], vbs[q], cus[q])
                scs.append(pltpu.async_copy(
                    cus[q], v_sh.at[db.at[gbase + q]], csem[q], add=True))
            for q in range(4):
                scs[q].wait()

        pltpu.sync_copy(bb, b_out.at[w])
        pltpu.sync_copy(sloc, s_sh.at[sid])
        plsc.subcore_barrier()

        rps = NROW // NS
        for j in range(rps):
            acc[j, :] = jnp.zeros((16,), jnp.float32)

        @pl.loop(0, NS)
        def _(j):
            pltpu.sync_copy(s_sh.at[j, pl.ds(sid * rps, rps)], tmp)
            for r in range(rps):
                acc[r, :] = acc[r, :] + tmp[r, :]

        pltpu.sync_copy(acc, s_part.at[cid, pl.ds(sid * rps, rps)])
        for z in range(NPS // ZCH):
            sl = pl.ds(sid * NPS + z * ZCH, ZCH)
            pltpu.sync_copy(v_sh.at[sl], vraw_part.at[cid, sl])

    return bpass


_sc_bpass_first = _make_sc_bpass(True)
_sc_bpass_later = _make_sc_bpass(False)


def kernel(x, edge_index, gcn_W, gcn_b, pre_W, pre_b, ln_g, ln_b, rt_W, rt_b,
           cls_W, cls_b):
    src = edge_index[0]
    dst = edge_index[1]
    padi = jnp.full((E_PAD - E,), N_PAD - 1, dtype=src.dtype)
    srcp = jnp.concatenate([src, padi]).reshape(NW, NG, G).astype(jnp.int32)
    dstp = jnp.concatenate([dst, padi]).reshape(NW, NG, G).astype(jnp.int32)

    xp = jnp.pad(x, ((0, N_PAD - N), (0, 0)))
    hW = _tc_matmul(xp, gcn_W)

    deg_part = _sc_deg(dstp)
    s0, g_tab, g2_tab, dinvb = _sc_nodeprep(deg_part, hW)

    hsum_part = _sc_gcn(srcp, dstp, g_tab)

    preWp = jnp.pad(pre_W, ((0, 0), (0, 128 - CAP)))
    prebp = jnp.pad(pre_b, (0, 128 - CAP)).reshape(1, 128)
    lngp = jnp.pad(ln_g, (0, 128 - CAP)).reshape(1, 128)
    lnbp = jnp.pad(ln_b, (0, 128 - CAP)).reshape(1, 128)
    rtWp = jnp.pad(rt_W, ((0, 128 - CAP), (0, 128 - CAP)))
    rtbp = jnp.pad(rt_b, (0, 128 - CAP)).reshape(1, 128)
    gbp = jnp.broadcast_to(gcn_b.reshape(1, H), (1, H))

    u_full = _tc_middle(hsum_part[0], hsum_part[1], dinvb, g2_tab, gbp,
                        preWp, prebp, lngp, lnbp, rtWp, rtbp)
    u32 = u_full[:, :CAP_PAD]

    dummy_b = jnp.zeros((NW, NG, G), jnp.float32)
    dummy_s = jnp.zeros((NROW, 16), jnp.float32)

    v_part = _sc_vpass0(srcp, dstp, u32)
    v = _sc_squash(v_part, s0, dummy_s)
    b, s_part, v_part = _sc_bpass_first(srcp, dstp, u32, v, dummy_b)
    v = _sc_squash(v_part, s_part[0], s_part[1])
    b, s_part, v_part = _sc_bpass_later(srcp, dstp, u32, v, b)
    v = _sc_squash(v_part, s_part[0], s_part[1])

    clsWp = jnp.pad(cls_W, ((0, CAP_PAD - CAP), (0, 128 - C)))
    clsbp = jnp.pad(cls_b, (0, 128 - C)).reshape(1, 128)
    out = pl.pallas_call(
        _mm_kernel,
        grid=(N_PAD // 1024,),
        in_specs=[pl.BlockSpec((1024, CAP_PAD), lambda i: (i, 0)),
                  pl.BlockSpec((CAP_PAD, 128), lambda i: (0, 0))],
        out_specs=pl.BlockSpec((1024, 128), lambda i: (i, 0)),
        out_shape=jax.ShapeDtypeStruct((N_PAD, 128), jnp.float32),
    )(v, clsWp)
    out = out + clsbp
    return out[:N, :C]


# speedup vs baseline: 20.9667x; 1.2658x over previous
import functools
import jax
import jax.numpy as jnp
import numpy as np
from jax import lax
from jax.experimental import pallas as pl
from jax.experimental.pallas import tpu as pltpu
from jax.experimental.pallas import tpu_sc as plsc

N = 10000
E = 320000
F_IN = 128
H = 64
CAP = 18
C = 7
ITERS = 3

NC, NS, L = 2, 16, 16
NW = NC * NS
N_PAD = 10240
E_PAD = 327680
G = 128
NG = E_PAD // (NW * G)
NROW = N_PAD // 16
CAP_PAD = 33

_SC_PARAMS = pltpu.CompilerParams(
    use_tc_tiling_on_sc=False, needs_layout_passes=False)
_MESH = plsc.VectorSubcoreMesh(core_axis_name="c", subcore_axis_name="s")

IOTA16 = lambda: lax.iota(jnp.int32, 16)


def _rsqrt_nt(x):
    bits = lax.bitcast_convert_type(x, jnp.int32)
    magic = jnp.full(x.shape, 0x5F3759DF, jnp.int32)
    y = lax.bitcast_convert_type(magic - lax.shift_right_arithmetic(bits, 1),
                                 jnp.float32)
    for _ in range(4):
        y = y * (1.5 - 0.5 * x * y * y)
    return y


def _wid():
    return lax.axis_index("s") * NC + lax.axis_index("c")


def _mm_kernel(x_ref, w_ref, o_ref):
    o_ref[...] = jnp.dot(x_ref[...], w_ref[...],
                         preferred_element_type=jnp.float32)


def _tc_matmul(x, w, bm=1024):
    m, k = x.shape
    _, n = w.shape
    return pl.pallas_call(
        _mm_kernel,
        grid=(m // bm,),
        in_specs=[pl.BlockSpec((bm, k), lambda i: (i, 0)),
                  pl.BlockSpec((k, n), lambda i: (0, 0))],
        out_specs=pl.BlockSpec((bm, n), lambda i: (i, 0)),
        out_shape=jax.ShapeDtypeStruct((m, n), jnp.float32),
    )(x, w)


def _mid_kernel(h0_ref, h1_ref, dinvb_ref, g2_ref, gb_ref, preW_ref, preb_ref,
                lng_ref, lnb_ref, rtW_ref, rtb_ref, o_ref):
    hs = h0_ref[...] + h1_ref[...]
    h = jax.nn.relu(dinvb_ref[...] * hs + g2_ref[...] + gb_ref[...])
    hp = jnp.dot(h, preW_ref[...], preferred_element_type=jnp.float32)
    hp = hp + preb_ref[...]
    mu = jnp.sum(hp, axis=-1, keepdims=True) * (1.0 / CAP)
    m2 = jnp.sum(hp * hp, axis=-1, keepdims=True) * (1.0 / CAP)
    var = m2 - mu * mu
    xln = (hp - mu) * lax.rsqrt(var + 1e-5) * lng_ref[...] + lnb_ref[...]
    u = jnp.dot(xln, rtW_ref[...], preferred_element_type=jnp.float32)
    o_ref[...] = u + rtb_ref[...]


def _tc_middle(h0, h1, dinvb, g2, gb, preW, preb, lng, lnb, rtW, rtb, bm=1024):
    m = h0.shape[0]
    row64 = lambda i: (i, 0)
    fixed = lambda i: (0, 0)
    return pl.pallas_call(
        _mid_kernel,
        grid=(m // bm,),
        in_specs=[
            pl.BlockSpec((bm, H), row64), pl.BlockSpec((bm, H), row64),
            pl.BlockSpec((bm, H), row64), pl.BlockSpec((bm, H), row64),
            pl.BlockSpec((1, H), fixed),
            pl.BlockSpec((H, 128), fixed), pl.BlockSpec((1, 128), fixed),
            pl.BlockSpec((1, 128), fixed), pl.BlockSpec((1, 128), fixed),
            pl.BlockSpec((128, 128), fixed), pl.BlockSpec((1, 128), fixed),
        ],
        out_specs=pl.BlockSpec((bm, 128), row64),
        out_shape=jax.ShapeDtypeStruct((m, 128), jnp.float32),
    )(h0, h1, dinvb, g2, gb, preW, preb, lng, lnb, rtW, rtb)


@functools.partial(
    pl.kernel,
    out_type=jax.ShapeDtypeStruct((NW, NROW, 17), jnp.float32),
    mesh=_MESH, compiler_params=_SC_PARAMS,
    scratch_types=[
        pltpu.VMEM((NG, G), jnp.int32),
        pltpu.VMEM((NROW, 17), jnp.float32),
    ],
)
def _sc_deg(dst_hbm, deg_part, dstb, degl):
    w = _wid()
    pltpu.sync_copy(dst_hbm.at[w], dstb)

    @pl.loop(0, NROW)
    def _(j):
        degl[j, pl.ds(0, 16)] = jnp.zeros((16,), jnp.float32)
        degl[j, pl.ds(1, 16)] = jnp.zeros((16,), jnp.float32)

    onev = jnp.ones((16,), jnp.float32)

    @pl.loop(0, NG)
    def _(g):
        for i in range(G // 16):
            idx = dstb[g, pl.ds(i * 16, 16)]
            plsc.addupdate_scatter(degl, [idx >> 4, idx & 15], onev)

    pltpu.sync_copy(degl, deg_part.at[w])


NPW = N_PAD // NW
RPW = NROW // NW

@functools.partial(
    pl.kernel,
    out_type=[
        jax.ShapeDtypeStruct((NROW, 16), jnp.float32),
        jax.ShapeDtypeStruct((N_PAD, H), jnp.float32),
        jax.ShapeDtypeStruct((N_PAD, H), jnp.float32),
        jax.ShapeDtypeStruct((N_PAD, H), jnp.float32),
    ],
    mesh=_MESH, compiler_params=_SC_PARAMS,
    scratch_types=[
        pltpu.VMEM((RPW, 16), jnp.float32),
        pltpu.VMEM((RPW, 17), jnp.float32),
        pltpu.VMEM((RPW, 16), jnp.float32),
        pltpu.VMEM((NPW, H), jnp.float32),
        pltpu.VMEM((NPW, H), jnp.float32),
        pltpu.VMEM((NPW, H), jnp.float32),
    ],
)
def _sc_nodeprep(deg_part, hw_hbm, s0_out, g_out, g2_out, dinvb_out,
                 acc, tmp, dnv, hbuf, h2buf, h3buf):
    w = _wid()
    for j in range(RPW):
        acc[j, :] = jnp.zeros((16,), jnp.float32)

    @pl.loop(0, NW)
    def _(w2):
        pltpu.sync_copy(deg_part.at[w2, pl.ds(w * RPW, RPW)], tmp)
        for j in range(RPW):
            acc[j, :] = acc[j, :] + tmp[j, pl.ds(0, 16)]

    for j in range(RPW):
        dnv[j, :] = _rsqrt_nt(acc[j, :] + 1.0)
    pltpu.sync_copy(acc, s0_out.at[pl.ds(w * RPW, RPW)])

    pltpu.sync_copy(hw_hbm.at[pl.ds(w * NPW, NPW)], hbuf)

    @pl.loop(0, NPW // 16)
    def _(t):
        n16 = IOTA16() + t * 16
        dvv = dnv[t, :]
        for k in range(H):
            kv = jnp.full((16,), k, jnp.int32)
            row = plsc.load_gather(hbuf, [n16, kv])
            gg = row * dvv
            plsc.store_scatter(hbuf, [n16, kv], gg)
            plsc.store_scatter(h2buf, [n16, kv], gg * dvv)
            plsc.store_scatter(h3buf, [n16, kv], dvv)

    pltpu.sync_copy(hbuf, g_out.at[pl.ds(w * NPW, NPW)])
    pltpu.sync_copy(h2buf, g2_out.at[pl.ds(w * NPW, NPW)])
    pltpu.sync_copy(h3buf, dinvb_out.at[pl.ds(w * NPW, NPW)])


NPS = N_PAD // NS
ZCH = 160

def _make_dma_pass(W):
    @functools.partial(
        pl.kernel,
        out_type=jax.ShapeDtypeStruct((NC, N_PAD, W), jnp.float32),
        mesh=_MESH, compiler_params=_SC_PARAMS,
        scratch_types=[
            pltpu.VMEM((NG, G), jnp.int32),
            pltpu.VMEM((NG, G), jnp.int32),
            pltpu.VMEM((G, W), jnp.float32),
            pltpu.VMEM((G, W), jnp.float32),
            pltpu.VMEM((G, W), jnp.float32),
            pltpu.VMEM((G, W), jnp.float32),
            pltpu.VMEM((ZCH, W), jnp.float32),
            pltpu.VMEM_SHARED((N_PAD, W), jnp.float32),
        ] + [pltpu.SemaphoreType.DMA] * 8,
    )
    def dma_pass(src_hbm, dst_hbm, tab_hbm, acc_part,
                 sb, db, rows0, rows1, rows2, rows3, zb, acc_sh,
                 gs0, gs1, gs2, gs3, ss0, ss1, ss2, ss3):
        w = _wid()
        cid = lax.axis_index("c")
        sid = lax.axis_index("s")
        pltpu.sync_copy(src_hbm.at[w], sb)
        pltpu.sync_copy(dst_hbm.at[w], db)

        @pl.loop(0, ZCH)
        def _(j):
            for k in range(W // 16):
                zb[j, pl.ds(k * 16, 16)] = jnp.zeros((16,), jnp.float32)
            if W % 16:
                zb[j, pl.ds(W - 16, 16)] = jnp.zeros((16,), jnp.float32)

        for z in range(NPS // ZCH):
            pltpu.sync_copy(zb, acc_sh.at[pl.ds(sid * NPS + z * ZCH, ZCH)])
        plsc.subcore_barrier()

        rows = [rows0, rows1, rows2, rows3]
        gsems = [gs0, gs1, gs2, gs3]
        ssems = [ss0, ss1, ss2, ss3]

        @pl.loop(0, NG // 4)
        def _(t):
            gbase = t * 4
            cps = [pltpu.async_copy(tab_hbm.at[sb.at[gbase + q]], rows[q],
                                    gsems[q]) for q in range(4)]
            scs = []
            for q in range(4):
                cps[q].wait()
                scs.append(pltpu.async_copy(rows[q], acc_sh.at[db.at[gbase + q]],
                                            ssems[q], add=True))
            for q in range(4):
                scs[q].wait()

        plsc.subcore_barrier()
        for z in range(NPS // ZCH):
            sl = pl.ds(sid * NPS + z * ZCH, ZCH)
            pltpu.sync_copy(acc_sh.at[sl], acc_part.at[cid, sl])

    return dma_pass


_sc_gcn = _make_dma_pass(H)
_sc_vpass0 = _make_dma_pass(CAP_PAD)


def _make_squash(first, vw):
    stypes = [
        pltpu.VMEM((NPW, vw), jnp.float32),
        pltpu.VMEM((NPW, vw), jnp.float32),
        pltpu.VMEM((NPW, CAP), jnp.float32),
        pltpu.VMEM((RPW, 16), jnp.float32),
    ]
    if not first:
        stypes.append(pltpu.VMEM((RPW, 17), jnp.float32))

    @functools.partial(
        pl.kernel,
        out_type=jax.ShapeDtypeStruct((N_PAD, CAP), jnp.float32),
        mesh=_MESH, compiler_params=_SC_PARAMS,
        scratch_types=stypes,
    )
    def squash(v_part, s_in, v_out, va, vb, vo, sa, *rest):
        w = _wid()
        sl = pl.ds(w * NPW, NPW)
        rsl = pl.ds(w * RPW, RPW)
        pltpu.sync_copy(v_part.at[0, sl], va)
        pltpu.sync_copy(v_part.at[1, sl], vb)
        if first:
            pltpu.sync_copy(s_in.at[rsl], sa)
        else:
            (stmp,) = rest
            for j in range(RPW):
                sa[j, :] = jnp.zeros((16,), jnp.float32)

            @pl.loop(0, NW)
            def _(w2):
                pltpu.sync_copy(s_in.at[w2, rsl], stmp)
                for j in range(RPW):
                    sa[j, :] = sa[j, :] + stmp[j, pl.ds(0, 16)]

        @pl.loop(0, NPW // 16)
        def _(t):
            n16 = IOTA16() + t * 16
            inv = 1.0 / (sa[t, :] + 1e-16)
            nsq = jnp.zeros((16,), jnp.float32)
            xs = []
            for k in range(CAP):
                kv = jnp.full((16,), k, jnp.int32)
                x = (plsc.load_gather(va, [n16, kv])
                     + plsc.load_gather(vb, [n16, kv])) * inv
                xs.append(x)
                nsq = nsq + x * x
            f = nsq / (1.0 + nsq) * _rsqrt_nt(nsq + 1e-9)
            for k in range(CAP):
                kv = jnp.full((16,), k, jnp.int32)
                plsc.store_scatter(vo, [n16, kv], xs[k] * f)

        pltpu.sync_copy(vo, v_out.at[sl])

    return squash


_sc_squash_first = _make_squash(True, CAP_PAD)
_sc_squash_later = _make_squash(False, CAP)


def _make_sc_bpass(first):
    @functools.partial(
        pl.kernel,
        out_type=[
            jax.ShapeDtypeStruct((NW, NG, G), jnp.float32),
            jax.ShapeDtypeStruct((NW, NROW, 17), jnp.float32),
            jax.ShapeDtypeStruct((NC, N_PAD, CAP), jnp.float32),
        ],
        mesh=_MESH, compiler_params=_SC_PARAMS,
        scratch_types=[
            pltpu.VMEM((NG, G), jnp.int32),
            pltpu.VMEM((NG, G), jnp.int32),
            pltpu.VMEM((NG, G), jnp.float32),
            pltpu.VMEM((NROW, 17), jnp.float32),
            pltpu.VMEM((G, CAP_PAD), jnp.float32),
            pltpu.VMEM((G, CAP), jnp.float32),
            pltpu.VMEM((G, CAP_PAD), jnp.float32),
            pltpu.VMEM((G, CAP), jnp.float32),
            pltpu.VMEM((G, CAP_PAD), jnp.float32),
            pltpu.VMEM((G, CAP), jnp.float32),
            pltpu.VMEM((G, CAP_PAD), jnp.float32),
            pltpu.VMEM((G, CAP), jnp.float32),
            pltpu.VMEM((G, CAP), jnp.float32),
            pltpu.VMEM((G, CAP), jnp.float32),
            pltpu.VMEM((G, CAP), jnp.float32),
            pltpu.VMEM((G, CAP), jnp.float32),
            pltpu.VMEM((ZCH, CAP), jnp.float32),
            pltpu.VMEM_SHARED((N_PAD, CAP), jnp.float32),
        ] + [pltpu.SemaphoreType.DMA] * 12,
    )
    def bpass(src_hbm, dst_hbm, u_hbm, v_hbm, b_in, b_out, s_part, vraw_part,
              sb, db, bb, sloc, ub0, vb0, ub1, vb1, ub2, vb2, ub3, vb3,
              cu0, cu1, cu2, cu3, zb,
              v_sh, us0, us1, us2, us3, vs0, vs1, vs2, vs3,
              cs0, cs1, cs2, cs3):
        w = _wid()
        cid = lax.axis_index("c")
        sid = lax.axis_index("s")
        pltpu.sync_copy(src_hbm.at[w], sb)
        pltpu.sync_copy(dst_hbm.at[w], db)
        if not first:
            pltpu.sync_copy(b_in.at[w], bb)

        @pl.loop(0, NROW)
        def _(j):
            sloc[j, pl.ds(0, 16)] = jnp.zeros((16,), jnp.float32)
            sloc[j, pl.ds(1, 16)] = jnp.zeros((16,), jnp.float32)

        @pl.loop(0, ZCH)
        def _(j):
            zb[j, pl.ds(0, 16)] = jnp.zeros((16,), jnp.float32)
            zb[j, pl.ds(2, 16)] = jnp.zeros((16,), jnp.float32)

        for z in range(NPS // ZCH):
            pltpu.sync_copy(zb, v_sh.at[pl.ds(sid * NPS + z * ZCH, ZCH)])
        plsc.subcore_barrier()

        def compute(g, ub, vb, cu):
            for i in range(G // 16):
                ev = IOTA16() + (i * 16)
                accs = [jnp.zeros((16,), jnp.float32) for _ in range(4)]
                uks = []
                for k in range(CAP):
                    kv = jnp.full((16,), k, jnp.int32)
                    uk = plsc.load_gather(ub, [ev, kv])
                    vk = plsc.load_gather(vb, [ev, kv])
                    uks.append(uk)
                    accs[k % 4] = accs[k % 4] + uk * vk
                accv = (accs[0] + accs[1]) + (accs[2] + accs[3])
                if first:
                    bnew = accv
                else:
                    bnew = bb[g, pl.ds(i * 16, 16)] + accv
                bb[g, pl.ds(i * 16, 16)] = bnew
                ex = jnp.exp(bnew)
                dv = db[g, pl.ds(i * 16, 16)]
                plsc.addupdate_scatter(sloc, [dv >> 4, dv & 15], ex)
                for k in range(CAP):
                    kv = jnp.full((16,), k, jnp.int32)
                    plsc.store_scatter(cu, [ev, kv], uks[k] * ex)

        ubs = [ub0, ub1, ub2, ub3]
        vbs = [vb0, vb1, vb2, vb3]
        cus = [cu0, cu1, cu2, cu3]
        usem = [us0, us1, us2, us3]
        vsem = [vs0, vs1, vs2, vs3]
        csem = [cs0, cs1, cs2, cs3]

        @pl.loop(0, NG // 4)
        def _(t):
            gbase = t * 4
            dus = [pltpu.async_copy(u_hbm.at[sb.at[gbase + q]], ubs[q], usem[q])
                   for q in range(4)]
            dvs = [pltpu.async_copy(v_hbm.at[db.at[gbase + q]], vbs[q], vsem[q])
                   for q in range(4)]
            scs = []
            for q in range(4):
                dus[q].wait()
                dvs[q].wait()
                compute(gbase + q, ubs[q], vbs[q], cus[q])
                scs.append(pltpu.async_copy(
                    cus[q], v_sh.at[db.at[gbase + q]], csem[q], add=True))
            for q in range(4):
                scs[q].wait()

        pltpu.sync_copy(bb, b_out.at[w])
        pltpu.sync_copy(sloc, s_part.at[w])
        plsc.subcore_barrier()
        for z in range(NPS // ZCH):
            sl = pl.ds(sid * NPS + z * ZCH, ZCH)
            pltpu.sync_copy(v_sh.at[sl], vraw_part.at[cid, sl])

    return bpass


_sc_bpass_first = _make_sc_bpass(True)
_sc_bpass_later = _make_sc_bpass(False)


def kernel(x, edge_index, gcn_W, gcn_b, pre_W, pre_b, ln_g, ln_b, rt_W, rt_b,
           cls_W, cls_b):
    src = edge_index[0]
    dst = edge_index[1]
    padi = jnp.full((E_PAD - E,), N_PAD - 1, dtype=src.dtype)
    srcp = jnp.concatenate([src, padi]).reshape(NW, NG, G).astype(jnp.int32)
    dstp = jnp.concatenate([dst, padi]).reshape(NW, NG, G).astype(jnp.int32)

    xp = jnp.pad(x, ((0, N_PAD - N), (0, 0)))
    hW = _tc_matmul(xp, gcn_W)

    deg_part = _sc_deg(dstp)
    s0, g_tab, g2_tab, dinvb = _sc_nodeprep(deg_part, hW)

    hsum_part = _sc_gcn(srcp, dstp, g_tab)

    preWp = jnp.pad(pre_W, ((0, 0), (0, 128 - CAP)))
    prebp = jnp.pad(pre_b, (0, 128 - CAP)).reshape(1, 128)
    lngp = jnp.pad(ln_g, (0, 128 - CAP)).reshape(1, 128)
    lnbp = jnp.pad(ln_b, (0, 128 - CAP)).reshape(1, 128)
    rtWp = jnp.pad(rt_W, ((0, 128 - CAP), (0, 128 - CAP)))
    rtbp = jnp.pad(rt_b, (0, 128 - CAP)).reshape(1, 128)
    gbp = jnp.broadcast_to(gcn_b.reshape(1, H), (1, H))

    u_full = _tc_middle(hsum_part[0], hsum_part[1], dinvb, g2_tab, gbp,
                        preWp, prebp, lngp, lnbp, rtWp, rtbp)
    u32 = u_full[:, :CAP_PAD]

    dummy_b = jnp.zeros((NW, NG, G), jnp.float32)

    v_part = _sc_vpass0(srcp, dstp, u32)
    v = _sc_squash_first(v_part, s0)
    b, s_part, v_part = _sc_bpass_first(srcp, dstp, u32, v, dummy_b)
    v = _sc_squash_later(v_part, s_part)
    b, s_part, v_part = _sc_bpass_later(srcp, dstp, u32, v, b)
    v = _sc_squash_later(v_part, s_part)

    clsWp = jnp.pad(cls_W, ((0, 32 - CAP), (0, 128 - C)))
    clsbp = jnp.pad(cls_b, (0, 128 - C)).reshape(1, 128)
    v32 = jnp.pad(v, ((0, 0), (0, 32 - CAP)))
    out = pl.pallas_call(
        _mm_kernel,
        grid=(N_PAD // 1024,),
        in_specs=[pl.BlockSpec((1024, 32), lambda i: (i, 0)),
                  pl.BlockSpec((32, 128), lambda i: (0, 0))],
        out_specs=pl.BlockSpec((1024, 128), lambda i: (i, 0)),
        out_shape=jax.ShapeDtypeStruct((N_PAD, 128), jnp.float32),
    )(v32, clsWp)
    out = out + clsbp
    return out[:N, :C]
